# Initial kernel scaffold; baseline (speedup 1.0000x reference)
import jax
import jax.numpy as jnp
from jax.experimental import pallas as pl


def kernel(x, edge_index, batch, cell_features, W1, b1, W2, b2, W3, b3, Wd, bd, Wc1, bc1, Wc2, bc2, Wm1, bm1, Wm2, bm2, Wo, bo):
    raise NotImplementedError("write your pallas kernel here")


# speedup vs baseline: 8.4729x; 8.4729x over previous
import functools

import jax
import jax.numpy as jnp
from jax import lax
from jax.experimental import pallas as pl
import jax.experimental.pallas.tpu as pltpu
from jax.experimental.pallas import tpu_sc as plsc

N = 10000
E = 320000
B = 256
NTILES = 32
EP = 10240
EPAD = NTILES * EP
K = 128
G = EP // K
NACC = 10240
ZR = NACC // 16
DEGW = 16
RB = 1000
_F32 = jnp.float32
_HI = jax.lax.Precision.HIGHEST

def _dot(a, b):
    return jax.lax.dot_general(a, b, (((1,), (0,)), ((), ())),
                               precision=_HI, preferred_element_type=_F32)


@functools.lru_cache(maxsize=None)
def _make_edge_scatter(F):
    _mesh = plsc.VectorSubcoreMesh(core_axis_name="c", subcore_axis_name="s")

    @functools.partial(
        pl.kernel,
        out_type=(jax.ShapeDtypeStruct((NACC, F), _F32),
                  jax.ShapeDtypeStruct((NACC, F), _F32)),
        mesh=_mesh,
        scratch_types=[
            pltpu.VMEM((K,), jnp.int32),
            pltpu.VMEM((K,), jnp.int32),
            pltpu.VMEM((K, F), _F32),
            pltpu.VMEM_SHARED((NACC, F), _F32),
            pltpu.SemaphoreType.DMA,
        ],
        compiler_params=pltpu.CompilerParams(use_tc_tiling_on_sc=False),
    )
    def body(p_hbm, src_hbm, dst_hbm, zer_hbm, out0, out1,
             src_v, dst_v, rows_v, acc_sh, sem):
        c = lax.axis_index("c")
        s = lax.axis_index("s")
        rows = pl.ds(s * ZR, ZR)
        pltpu.sync_copy(zer_hbm, acc_sh.at[rows])
        plsc.subcore_barrier()
        base = (c * 16 + s) * EP

        def step(g, carry):
            off = pl.multiple_of(base + g * K, K)
            pltpu.sync_copy(src_hbm.at[pl.ds(off, K)], src_v)
            pltpu.sync_copy(dst_hbm.at[pl.ds(off, K)], dst_v)
            pltpu.async_copy(p_hbm.at[src_v], rows_v, sem).wait()
            pltpu.sync_copy(rows_v, acc_sh.at[dst_v], add=True)
            return carry

        lax.fori_loop(0, G, step, 0)
        plsc.subcore_barrier()

        @pl.when(c == 0)
        def _():
            pltpu.sync_copy(acc_sh.at[rows], out0.at[rows])

        @pl.when(c == 1)
        def _():
            pltpu.sync_copy(acc_sh.at[rows], out1.at[rows])

    return body


@functools.lru_cache(maxsize=None)
def _make_deg_scatter():
    _mesh = plsc.VectorSubcoreMesh(core_axis_name="c", subcore_axis_name="s")

    @functools.partial(
        pl.kernel,
        out_type=(jax.ShapeDtypeStruct((NACC, DEGW), _F32),
                  jax.ShapeDtypeStruct((NACC, DEGW), _F32)),
        mesh=_mesh,
        scratch_types=[
            pltpu.VMEM((K,), jnp.int32),
            pltpu.VMEM((K, DEGW), _F32),
            pltpu.VMEM_SHARED((NACC, DEGW), _F32),
        ],
        compiler_params=pltpu.CompilerParams(use_tc_tiling_on_sc=False),
    )
    def body(dst_hbm, ones_hbm, zer_hbm, out0, out1, dst_v, ones_v, acc_sh):
        c = lax.axis_index("c")
        s = lax.axis_index("s")
        rows = pl.ds(s * ZR, ZR)
        pltpu.sync_copy(zer_hbm, acc_sh.at[rows])
        pltpu.sync_copy(ones_hbm, ones_v)
        plsc.subcore_barrier()
        base = (c * 16 + s) * EP

        def step(g, carry):
            off = pl.multiple_of(base + g * K, K)
            pltpu.sync_copy(dst_hbm.at[pl.ds(off, K)], dst_v)
            pltpu.sync_copy(ones_v, acc_sh.at[dst_v], add=True)
            return carry

        lax.fori_loop(0, G, step, 0)
        plsc.subcore_barrier()

        @pl.when(c == 0)
        def _():
            pltpu.sync_copy(acc_sh.at[rows], out0.at[rows])

        @pl.when(c == 1)
        def _():
            pltpu.sync_copy(acc_sh.at[rows], out1.at[rows])

    return body


def _dinv_of(d0, d1):
    return jax.lax.rsqrt(d0[:, :1] + d1[:, :1] + 1.0)


def _tca_body(x_ref, w1_ref, d0_ref, d1_ref, p1_ref):
    di = _dinv_of(d0_ref[...], d1_ref[...])
    p1_ref[...] = di * _dot(x_ref[...], w1_ref[...])


def _tcb_body(a0_ref, a1_ref, p1_ref, b1_ref, d0_ref, d1_ref, p2_ref):
    di = _dinv_of(d0_ref[...], d1_ref[...])
    conv = di * (a0_ref[...] + a1_ref[...] + p1_ref[...]) + b1_ref[...]
    p2_ref[...] = di * jnp.maximum(conv, 0.0)


def _tcc_body(a0_ref, a1_ref, p2_ref, w2_ref, b2_ref, d0_ref, d1_ref, p3_ref):
    di = _dinv_of(d0_ref[...], d1_ref[...])
    m = di * (a0_ref[...] + a1_ref[...] + p2_ref[...])
    conv = _dot(m, w2_ref[...]) + b2_ref[...]
    p3_ref[...] = di * jnp.maximum(conv, 0.0)


def _tcd_body(a0_ref, a1_ref, p3_ref, w3_ref, b3_ref, d0_ref, d1_ref, h3_ref):
    di = _dinv_of(d0_ref[...], d1_ref[...])
    m = di * (a0_ref[...] + a1_ref[...] + p3_ref[...])
    h3_ref[...] = jnp.maximum(_dot(m, w3_ref[...]) + b3_ref[...], 0.0)


def _pool_body(batch_ref, h3_ref, psum_ref, cnt_ref):
    @pl.when(pl.program_id(0) == 0)
    def _():
        psum_ref[...] = jnp.zeros_like(psum_ref)
        cnt_ref[...] = jnp.zeros_like(cnt_ref)

    oh = (batch_ref[...] == jax.lax.broadcasted_iota(jnp.int32, (1, B), 1))
    oh = oh.astype(_F32)
    tdot = lambda a, b: jax.lax.dot_general(
        a, b, (((0,), (0,)), ((), ())), precision=_HI,
        preferred_element_type=_F32)
    psum_ref[...] += tdot(oh, h3_ref[...])
    cnt_ref[...] += tdot(oh, jnp.ones((RB, 1), _F32))


def _head_body(psum_ref, cnt_ref, cf_ref, wd_ref, bd_ref, wc1_ref, bc1_ref,
               wc2_ref, bc2_ref, wm1_ref, bm1_ref, wm2_ref, bm2_ref,
               wo_ref, bo_ref, out_ref):
    mean = psum_ref[...] / jnp.maximum(cnt_ref[...], 1.0)
    drug = _dot(mean, wd_ref[...]) + bd_ref[...]
    cellh = jnp.maximum(_dot(cf_ref[...], wc1_ref[...]) + bc1_ref[...], 0.0)
    cell = _dot(cellh, wc2_ref[...]) + bc2_ref[...]
    wm1 = wm1_ref[...]
    z = jnp.maximum(_dot(drug, wm1[:64]) + _dot(cell, wm1[64:])
                    + bm1_ref[...], 0.0)
    z = jnp.maximum(_dot(z, wm2_ref[...]) + bm2_ref[...], 0.0)
    out_ref[...] = _dot(z, wo_ref[...]) + bo_ref[...]


def _rows(shape):
    return pl.BlockSpec(shape, lambda i: (i, 0))


def _full(shape):
    return pl.BlockSpec(shape, lambda i: (0, 0))


_GRID = N // RB


def _tc_call(body, n_out_cols, in_specs):
    return pl.pallas_call(
        body,
        grid=(_GRID,),
        in_specs=in_specs,
        out_specs=_rows((RB, n_out_cols)),
        out_shape=jax.ShapeDtypeStruct((N, n_out_cols), _F32),
    )


def kernel(x, edge_index, batch, cell_features, W1, b1, W2, b2, W3, b3,
           Wd, bd, Wc1, bc1, Wc2, bc2, Wm1, bm1, Wm2, bm2, Wo, bo):
    src = edge_index[0].astype(jnp.int32)
    dst = edge_index[1].astype(jnp.int32)
    batch = batch.astype(jnp.int32)

    pad = EPAD - E
    srcp = jnp.concatenate([src, jnp.zeros((pad,), jnp.int32)])
    dstp = jnp.concatenate([dst, jnp.full((pad,), N, jnp.int32)])

    z16 = jnp.zeros((ZR, DEGW), _F32)
    z64 = jnp.zeros((ZR, 64), _F32)
    z128 = jnp.zeros((ZR, 128), _F32)
    ones16 = jnp.ones((K, DEGW), _F32)

    d0, d1 = _make_deg_scatter()(dstp, ones16, z16)

    degspec = [_rows((RB, DEGW)), _rows((RB, DEGW))]
    p1 = _tc_call(_tca_body, 64,
                  [_rows((RB, 128)), _full((128, 64))] + degspec)(
                      x, W1, d0, d1)

    a0, a1 = _make_edge_scatter(64)(p1, srcp, dstp, z64)
    p2 = _tc_call(_tcb_body, 64,
                  [_rows((RB, 64)), _rows((RB, 64)), _rows((RB, 64)),
                   _full((1, 64))] + degspec)(
                      a0, a1, p1, b1.reshape(1, 64), d0, d1)

    a0, a1 = _make_edge_scatter(64)(p2, srcp, dstp, z64)
    p3 = _tc_call(_tcc_body, 128,
                  [_rows((RB, 64)), _rows((RB, 64)), _rows((RB, 64)),
                   _full((64, 128)), _full((1, 128))] + degspec)(
                      a0, a1, p2, W2, b2.reshape(1, 128), d0, d1)

    a0, a1 = _make_edge_scatter(128)(p3, srcp, dstp, z128)
    h3 = _tc_call(_tcd_body, 256,
                  [_rows((RB, 128)), _rows((RB, 128)), _rows((RB, 128)),
                   _full((128, 256)), _full((1, 256))] + degspec)(
                      a0, a1, p3, W3, b3.reshape(1, 256), d0, d1)

    psum, cnt = pl.pallas_call(
        _pool_body,
        grid=(_GRID,),
        in_specs=[_rows((RB, 1)), _rows((RB, 256))],
        out_specs=(_full((B, 256)), _full((B, 1))),
        out_shape=(jax.ShapeDtypeStruct((B, 256), _F32),
                   jax.ShapeDtypeStruct((B, 1), _F32)),
    )(batch.reshape(N, 1), h3)

    out = pl.pallas_call(
        _head_body,
        out_shape=jax.ShapeDtypeStruct((B, 1), _F32),
    )(psum, cnt, cell_features, Wd, bd.reshape(1, 64), Wc1,
      bc1.reshape(1, 128), Wc2, bc2.reshape(1, 64), Wm1,
      bm1.reshape(1, 64), Wm2, bm2.reshape(1, 32), Wo, bo.reshape(1, 1))

    return out.reshape(-1)


# speedup vs baseline: 12.1940x; 1.4392x over previous
import functools

import jax
import jax.numpy as jnp
from jax import lax
from jax.experimental import pallas as pl
import jax.experimental.pallas.tpu as pltpu
from jax.experimental.pallas import tpu_sc as plsc

N = 10000
E = 320000
B = 256
NTILES = 32
EP = 10240
EPAD = NTILES * EP
K = 80
G = EP // K
NACC = 10016
ZR = NACC // 16
DEGW = 8
RB = 1000
_F32 = jnp.float32
_HI = jax.lax.Precision.HIGHEST

def _dot(a, b):
    return jax.lax.dot_general(a, b, (((1,), (0,)), ((), ())),
                               precision=_HI, preferred_element_type=_F32)


@functools.lru_cache(maxsize=None)
def _make_edge_scatter(F):
    _mesh = plsc.VectorSubcoreMesh(core_axis_name="c", subcore_axis_name="s")

    @functools.partial(
        pl.kernel,
        out_type=(jax.ShapeDtypeStruct((NACC, F), _F32),
                  jax.ShapeDtypeStruct((NACC, F), _F32)),
        mesh=_mesh,
        scratch_types=[
            pltpu.VMEM((G, K), jnp.int32),
            pltpu.VMEM((G, K), jnp.int32),
            pltpu.VMEM((K, F), _F32),
            pltpu.VMEM((K, F), _F32),
            pltpu.VMEM_SHARED((NACC, F), _F32),
            pltpu.SemaphoreType.DMA,
            pltpu.SemaphoreType.DMA,
        ],
        compiler_params=pltpu.CompilerParams(use_tc_tiling_on_sc=False),
    )
    def body(p_hbm, src_hbm, dst_hbm, zer_hbm, out0, out1,
             src_all, dst_all, rows0, rows1, acc_sh, sem0, sem1):
        c = lax.axis_index("c")
        s = lax.axis_index("s")
        rows = pl.ds(s * ZR, ZR)
        pltpu.sync_copy(zer_hbm, acc_sh.at[rows])
        wid = c * 16 + s
        pltpu.sync_copy(src_hbm.at[pl.ds(wid * G, G)], src_all)
        pltpu.sync_copy(dst_hbm.at[pl.ds(wid * G, G)], dst_all)
        plsc.subcore_barrier()

        pltpu.async_copy(p_hbm.at[src_all.at[0]], rows0, sem0)
        pltpu.async_copy(p_hbm.at[src_all.at[1]], rows1, sem1)

        def pair(j, carry):
            g0 = j * 2
            pltpu.make_async_copy(p_hbm.at[src_all.at[g0]], rows0, sem0).wait()
            pltpu.sync_copy(rows0, acc_sh.at[dst_all.at[g0]], add=True)

            @pl.when(j < G // 2 - 1)
            def _():
                pltpu.async_copy(p_hbm.at[src_all.at[g0 + 2]], rows0, sem0)

            pltpu.make_async_copy(p_hbm.at[src_all.at[g0 + 1]], rows1,
                                  sem1).wait()
            pltpu.sync_copy(rows1, acc_sh.at[dst_all.at[g0 + 1]], add=True)

            @pl.when(j < G // 2 - 1)
            def _():
                pltpu.async_copy(p_hbm.at[src_all.at[g0 + 3]], rows1, sem1)

            return carry

        lax.fori_loop(0, G // 2, pair, 0)
        plsc.subcore_barrier()

        @pl.when(c == 0)
        def _():
            pltpu.sync_copy(acc_sh.at[rows], out0.at[rows])

        @pl.when(c == 1)
        def _():
            pltpu.sync_copy(acc_sh.at[rows], out1.at[rows])

    return body


@functools.lru_cache(maxsize=None)
def _make_deg_scatter():
    _mesh = plsc.VectorSubcoreMesh(core_axis_name="c", subcore_axis_name="s")

    @functools.partial(
        pl.kernel,
        out_type=(jax.ShapeDtypeStruct((NACC, DEGW), _F32),
                  jax.ShapeDtypeStruct((NACC, DEGW), _F32)),
        mesh=_mesh,
        scratch_types=[
            pltpu.VMEM((G, K), jnp.int32),
            pltpu.VMEM((K, DEGW), _F32),
            pltpu.VMEM_SHARED((NACC, DEGW), _F32),
        ],
        compiler_params=pltpu.CompilerParams(use_tc_tiling_on_sc=False),
    )
    def body(dst_hbm, ones_hbm, zer_hbm, out0, out1, dst_all, ones_v, acc_sh):
        c = lax.axis_index("c")
        s = lax.axis_index("s")
        rows = pl.ds(s * ZR, ZR)
        pltpu.sync_copy(zer_hbm, acc_sh.at[rows])
        pltpu.sync_copy(ones_hbm, ones_v)
        wid = c * 16 + s
        pltpu.sync_copy(dst_hbm.at[pl.ds(wid * G, G)], dst_all)
        plsc.subcore_barrier()

        def step(g, carry):
            pltpu.sync_copy(ones_v, acc_sh.at[dst_all.at[g]], add=True)
            return carry

        lax.fori_loop(0, G, step, 0)
        plsc.subcore_barrier()

        @pl.when(c == 0)
        def _():
            pltpu.sync_copy(acc_sh.at[rows], out0.at[rows])

        @pl.when(c == 1)
        def _():
            pltpu.sync_copy(acc_sh.at[rows], out1.at[rows])

    return body


def _dinv_of(d0, d1):
    return jax.lax.rsqrt(d0[:, :1] + d1[:, :1] + 1.0)


def _tca_body(x_ref, w1_ref, d0_ref, d1_ref, p1_ref):
    di = _dinv_of(d0_ref[...], d1_ref[...])
    p1_ref[...] = di * _dot(x_ref[...], w1_ref[...])


def _tcb_body(a0_ref, a1_ref, p1_ref, b1_ref, d0_ref, d1_ref, p2_ref):
    di = _dinv_of(d0_ref[...], d1_ref[...])
    conv = di * (a0_ref[...] + a1_ref[...] + p1_ref[...]) + b1_ref[...]
    p2_ref[...] = di * jnp.maximum(conv, 0.0)


def _tcc_body(a0_ref, a1_ref, p2_ref, w2_ref, b2_ref, d0_ref, d1_ref, p3_ref):
    di = _dinv_of(d0_ref[...], d1_ref[...])
    m = di * (a0_ref[...] + a1_ref[...] + p2_ref[...])
    conv = _dot(m, w2_ref[...]) + b2_ref[...]
    p3_ref[...] = di * jnp.maximum(conv, 0.0)


def _tcd_body(a0_ref, a1_ref, p3_ref, w3_ref, b3_ref, d0_ref, d1_ref, h3_ref):
    di = _dinv_of(d0_ref[...], d1_ref[...])
    m = di * (a0_ref[...] + a1_ref[...] + p3_ref[...])
    h3_ref[...] = jnp.maximum(_dot(m, w3_ref[...]) + b3_ref[...], 0.0)


def _pool_body(batch_ref, h3_ref, psum_ref, cnt_ref):
    @pl.when(pl.program_id(0) == 0)
    def _():
        psum_ref[...] = jnp.zeros_like(psum_ref)
        cnt_ref[...] = jnp.zeros_like(cnt_ref)

    oh = (batch_ref[...] == jax.lax.broadcasted_iota(jnp.int32, (1, B), 1))
    oh = oh.astype(_F32)
    tdot = lambda a, b: jax.lax.dot_general(
        a, b, (((0,), (0,)), ((), ())), precision=_HI,
        preferred_element_type=_F32)
    psum_ref[...] += tdot(oh, h3_ref[...])
    cnt_ref[...] += tdot(oh, jnp.ones((RB, 1), _F32))


def _head_body(psum_ref, cnt_ref, cf_ref, wd_ref, bd_ref, wc1_ref, bc1_ref,
               wc2_ref, bc2_ref, wm1_ref, bm1_ref, wm2_ref, bm2_ref,
               wo_ref, bo_ref, out_ref):
    mean = psum_ref[...] / jnp.maximum(cnt_ref[...], 1.0)
    drug = _dot(mean, wd_ref[...]) + bd_ref[...]
    cellh = jnp.maximum(_dot(cf_ref[...], wc1_ref[...]) + bc1_ref[...], 0.0)
    cell = _dot(cellh, wc2_ref[...]) + bc2_ref[...]
    wm1 = wm1_ref[...]
    z = jnp.maximum(_dot(drug, wm1[:64]) + _dot(cell, wm1[64:])
                    + bm1_ref[...], 0.0)
    z = jnp.maximum(_dot(z, wm2_ref[...]) + bm2_ref[...], 0.0)
    out_ref[...] = _dot(z, wo_ref[...]) + bo_ref[...]


def _rows(shape):
    return pl.BlockSpec(shape, lambda i: (i, 0))


def _full(shape):
    return pl.BlockSpec(shape, lambda i: (0, 0))


_GRID = N // RB


def _tc_call(body, n_out_cols, in_specs):
    return pl.pallas_call(
        body,
        grid=(_GRID,),
        in_specs=in_specs,
        out_specs=_rows((RB, n_out_cols)),
        out_shape=jax.ShapeDtypeStruct((N, n_out_cols), _F32),
    )


def kernel(x, edge_index, batch, cell_features, W1, b1, W2, b2, W3, b3,
           Wd, bd, Wc1, bc1, Wc2, bc2, Wm1, bm1, Wm2, bm2, Wo, bo):
    src = edge_index[0].astype(jnp.int32)
    dst = edge_index[1].astype(jnp.int32)
    batch = batch.astype(jnp.int32)

    pad = EPAD - E
    srcp = jnp.concatenate([src, jnp.zeros((pad,), jnp.int32)])
    dstp = jnp.concatenate([dst, jnp.full((pad,), N, jnp.int32)])
    srcp = srcp.reshape(NTILES * G, K)
    dstp = dstp.reshape(NTILES * G, K)

    z16 = jnp.zeros((ZR, DEGW), _F32)
    z64 = jnp.zeros((ZR, 64), _F32)
    z128 = jnp.zeros((ZR, 128), _F32)
    ones16 = jnp.ones((K, DEGW), _F32)

    d0, d1 = _make_deg_scatter()(dstp, ones16, z16)

    degspec = [_rows((RB, DEGW)), _rows((RB, DEGW))]
    p1 = _tc_call(_tca_body, 64,
                  [_rows((RB, 128)), _full((128, 64))] + degspec)(
                      x, W1, d0, d1)

    a0, a1 = _make_edge_scatter(64)(p1, srcp, dstp, z64)
    p2 = _tc_call(_tcb_body, 64,
                  [_rows((RB, 64)), _rows((RB, 64)), _rows((RB, 64)),
                   _full((1, 64))] + degspec)(
                      a0, a1, p1, b1.reshape(1, 64), d0, d1)

    a0, a1 = _make_edge_scatter(64)(p2, srcp, dstp, z64)
    p3 = _tc_call(_tcc_body, 128,
                  [_rows((RB, 64)), _rows((RB, 64)), _rows((RB, 64)),
                   _full((64, 128)), _full((1, 128))] + degspec)(
                      a0, a1, p2, W2, b2.reshape(1, 128), d0, d1)

    a0, a1 = _make_edge_scatter(128)(p3, srcp, dstp, z128)
    h3 = _tc_call(_tcd_body, 256,
                  [_rows((RB, 128)), _rows((RB, 128)), _rows((RB, 128)),
                   _full((128, 256)), _full((1, 256))] + degspec)(
                      a0, a1, p3, W3, b3.reshape(1, 256), d0, d1)

    psum, cnt = pl.pallas_call(
        _pool_body,
        grid=(_GRID,),
        in_specs=[_rows((RB, 1)), _rows((RB, 256))],
        out_specs=(_full((B, 256)), _full((B, 1))),
        out_shape=(jax.ShapeDtypeStruct((B, 256), _F32),
                   jax.ShapeDtypeStruct((B, 1), _F32)),
    )(batch.reshape(N, 1), h3)

    out = pl.pallas_call(
        _head_body,
        out_shape=jax.ShapeDtypeStruct((B, 1), _F32),
    )(psum, cnt, cell_features, Wd, bd.reshape(1, 64), Wc1,
      bc1.reshape(1, 128), Wc2, bc2.reshape(1, 64), Wm1,
      bm1.reshape(1, 64), Wm2, bm2.reshape(1, 32), Wo, bo.reshape(1, 1))

    return out.reshape(-1)


# speedup vs baseline: 12.4261x; 1.0190x over previous
import functools

import jax
import jax.numpy as jnp
from jax import lax
from jax.experimental import pallas as pl
import jax.experimental.pallas.tpu as pltpu
from jax.experimental.pallas import tpu_sc as plsc

N = 10000
E = 320000
B = 256
NTILES = 32
EP = 10240
EPAD = NTILES * EP
K = 80
G = EP // K
NACC = 10016
ZR = NACC // 16
DEGW = 8
RB = 1000
_F32 = jnp.float32
_HI = jax.lax.Precision.DEFAULT

def _dot(a, b):
    return jax.lax.dot_general(a, b, (((1,), (0,)), ((), ())),
                               precision=_HI, preferred_element_type=_F32)


@functools.lru_cache(maxsize=None)
def _make_edge_scatter(F):
    _mesh = plsc.VectorSubcoreMesh(core_axis_name="c", subcore_axis_name="s")

    @functools.partial(
        pl.kernel,
        out_type=(jax.ShapeDtypeStruct((NACC, F), _F32),
                  jax.ShapeDtypeStruct((NACC, F), _F32)),
        mesh=_mesh,
        scratch_types=[
            pltpu.VMEM((G, K), jnp.int32),
            pltpu.VMEM((G, K), jnp.int32),
            pltpu.VMEM((K, F), _F32),
            pltpu.VMEM((K, F), _F32),
            pltpu.VMEM_SHARED((NACC, F), _F32),
            pltpu.SemaphoreType.DMA,
            pltpu.SemaphoreType.DMA,
        ],
        compiler_params=pltpu.CompilerParams(use_tc_tiling_on_sc=False),
    )
    def body(p_hbm, src_hbm, dst_hbm, zer_hbm, out0, out1,
             src_all, dst_all, rows0, rows1, acc_sh, sem0, sem1):
        c = lax.axis_index("c")
        s = lax.axis_index("s")
        rows = pl.ds(s * ZR, ZR)
        pltpu.sync_copy(zer_hbm, acc_sh.at[rows])
        wid = c * 16 + s
        pltpu.sync_copy(src_hbm.at[pl.ds(wid * G, G)], src_all)
        pltpu.sync_copy(dst_hbm.at[pl.ds(wid * G, G)], dst_all)
        plsc.subcore_barrier()

        pltpu.async_copy(p_hbm.at[src_all.at[0]], rows0, sem0)
        pltpu.async_copy(p_hbm.at[src_all.at[1]], rows1, sem1)

        def pair(j, carry):
            g0 = j * 2
            pltpu.make_async_copy(p_hbm.at[src_all.at[g0]], rows0, sem0).wait()
            pltpu.sync_copy(rows0, acc_sh.at[dst_all.at[g0]], add=True)

            @pl.when(j < G // 2 - 1)
            def _():
                pltpu.async_copy(p_hbm.at[src_all.at[g0 + 2]], rows0, sem0)

            pltpu.make_async_copy(p_hbm.at[src_all.at[g0 + 1]], rows1,
                                  sem1).wait()
            pltpu.sync_copy(rows1, acc_sh.at[dst_all.at[g0 + 1]], add=True)

            @pl.when(j < G // 2 - 1)
            def _():
                pltpu.async_copy(p_hbm.at[src_all.at[g0 + 3]], rows1, sem1)

            return carry

        lax.fori_loop(0, G // 2, pair, 0)
        plsc.subcore_barrier()

        @pl.when(c == 0)
        def _():
            pltpu.sync_copy(acc_sh.at[rows], out0.at[rows])

        @pl.when(c == 1)
        def _():
            pltpu.sync_copy(acc_sh.at[rows], out1.at[rows])

    return body


@functools.lru_cache(maxsize=None)
def _make_deg_scatter():
    _mesh = plsc.VectorSubcoreMesh(core_axis_name="c", subcore_axis_name="s")

    @functools.partial(
        pl.kernel,
        out_type=(jax.ShapeDtypeStruct((NACC, DEGW), _F32),
                  jax.ShapeDtypeStruct((NACC, DEGW), _F32)),
        mesh=_mesh,
        scratch_types=[
            pltpu.VMEM((G, K), jnp.int32),
            pltpu.VMEM((K, DEGW), _F32),
            pltpu.VMEM_SHARED((NACC, DEGW), _F32),
        ],
        compiler_params=pltpu.CompilerParams(use_tc_tiling_on_sc=False),
    )
    def body(dst_hbm, ones_hbm, zer_hbm, out0, out1, dst_all, ones_v, acc_sh):
        c = lax.axis_index("c")
        s = lax.axis_index("s")
        rows = pl.ds(s * ZR, ZR)
        pltpu.sync_copy(zer_hbm, acc_sh.at[rows])
        pltpu.sync_copy(ones_hbm, ones_v)
        wid = c * 16 + s
        pltpu.sync_copy(dst_hbm.at[pl.ds(wid * G, G)], dst_all)
        plsc.subcore_barrier()

        def step(g, carry):
            pltpu.sync_copy(ones_v, acc_sh.at[dst_all.at[g]], add=True)
            return carry

        lax.fori_loop(0, G, step, 0)
        plsc.subcore_barrier()

        @pl.when(c == 0)
        def _():
            pltpu.sync_copy(acc_sh.at[rows], out0.at[rows])

        @pl.when(c == 1)
        def _():
            pltpu.sync_copy(acc_sh.at[rows], out1.at[rows])

    return body


def _dinv_of(d0, d1):
    return 1.0 / jnp.sqrt(d0[:, :1] + d1[:, :1] + 1.0)


def _tca_body(x_ref, w1_ref, d0_ref, d1_ref, p1_ref):
    di = _dinv_of(d0_ref[...], d1_ref[...])
    p1_ref[...] = di * _dot(x_ref[...], w1_ref[...])


def _tcb_body(a0_ref, a1_ref, p1_ref, b1_ref, d0_ref, d1_ref, p2_ref):
    di = _dinv_of(d0_ref[...], d1_ref[...])
    conv = di * (a0_ref[...] + a1_ref[...] + p1_ref[...]) + b1_ref[...]
    p2_ref[...] = di * jnp.maximum(conv, 0.0)


def _tcc_body(a0_ref, a1_ref, p2_ref, w2_ref, b2_ref, d0_ref, d1_ref, p3_ref):
    di = _dinv_of(d0_ref[...], d1_ref[...])
    m = di * (a0_ref[...] + a1_ref[...] + p2_ref[...])
    conv = _dot(m, w2_ref[...]) + b2_ref[...]
    p3_ref[...] = di * jnp.maximum(conv, 0.0)


def _tcd_body(a0_ref, a1_ref, p3_ref, w3_ref, b3_ref, d0_ref, d1_ref, h3_ref):
    di = _dinv_of(d0_ref[...], d1_ref[...])
    m = di * (a0_ref[...] + a1_ref[...] + p3_ref[...])
    h3_ref[...] = jnp.maximum(_dot(m, w3_ref[...]) + b3_ref[...], 0.0)


def _pool_body(batch_ref, h3_ref, psum_ref, cnt_ref):
    @pl.when(pl.program_id(0) == 0)
    def _():
        psum_ref[...] = jnp.zeros_like(psum_ref)
        cnt_ref[...] = jnp.zeros_like(cnt_ref)

    oh = (batch_ref[...] == jax.lax.broadcasted_iota(jnp.int32, (1, B), 1))
    oh = oh.astype(_F32)
    tdot = lambda a, b: jax.lax.dot_general(
        a, b, (((0,), (0,)), ((), ())), precision=_HI,
        preferred_element_type=_F32)
    psum_ref[...] += tdot(oh, h3_ref[...])
    cnt_ref[...] += tdot(oh, jnp.ones((RB, 1), _F32))


def _head_body(psum_ref, cnt_ref, cf_ref, wd_ref, bd_ref, wc1_ref, bc1_ref,
               wc2_ref, bc2_ref, wm1_ref, bm1_ref, wm2_ref, bm2_ref,
               wo_ref, bo_ref, out_ref):
    mean = psum_ref[...] / jnp.maximum(cnt_ref[...], 1.0)
    drug = _dot(mean, wd_ref[...]) + bd_ref[...]
    cellh = jnp.maximum(_dot(cf_ref[...], wc1_ref[...]) + bc1_ref[...], 0.0)
    cell = _dot(cellh, wc2_ref[...]) + bc2_ref[...]
    wm1 = wm1_ref[...]
    z = jnp.maximum(_dot(drug, wm1[:64]) + _dot(cell, wm1[64:])
                    + bm1_ref[...], 0.0)
    z = jnp.maximum(_dot(z, wm2_ref[...]) + bm2_ref[...], 0.0)
    out_ref[...] = _dot(z, wo_ref[...]) + bo_ref[...]


def _rows(shape):
    return pl.BlockSpec(shape, lambda i: (i, 0))


def _full(shape):
    return pl.BlockSpec(shape, lambda i: (0, 0))


_GRID = N // RB


def _tc_call(body, n_out_cols, in_specs):
    return pl.pallas_call(
        body,
        grid=(_GRID,),
        in_specs=in_specs,
        out_specs=_rows((RB, n_out_cols)),
        out_shape=jax.ShapeDtypeStruct((N, n_out_cols), _F32),
    )


def kernel(x, edge_index, batch, cell_features, W1, b1, W2, b2, W3, b3,
           Wd, bd, Wc1, bc1, Wc2, bc2, Wm1, bm1, Wm2, bm2, Wo, bo):
    src = edge_index[0].astype(jnp.int32)
    dst = edge_index[1].astype(jnp.int32)
    batch = batch.astype(jnp.int32)

    pad = EPAD - E
    srcp = jnp.concatenate([src, jnp.zeros((pad,), jnp.int32)])
    dstp = jnp.concatenate([dst, jnp.full((pad,), N, jnp.int32)])
    srcp = srcp.reshape(NTILES * G, K)
    dstp = dstp.reshape(NTILES * G, K)

    z16 = jnp.zeros((ZR, DEGW), _F32)
    z64 = jnp.zeros((ZR, 64), _F32)
    z128 = jnp.zeros((ZR, 128), _F32)
    ones16 = jnp.ones((K, DEGW), _F32)

    d0, d1 = _make_deg_scatter()(dstp, ones16, z16)

    degspec = [_rows((RB, DEGW)), _rows((RB, DEGW))]
    p1 = _tc_call(_tca_body, 64,
                  [_rows((RB, 128)), _full((128, 64))] + degspec)(
                      x, W1, d0, d1)

    a0, a1 = _make_edge_scatter(64)(p1, srcp, dstp, z64)
    p2 = _tc_call(_tcb_body, 64,
                  [_rows((RB, 64)), _rows((RB, 64)), _rows((RB, 64)),
                   _full((1, 64))] + degspec)(
                      a0, a1, p1, b1.reshape(1, 64), d0, d1)

    a0, a1 = _make_edge_scatter(64)(p2, srcp, dstp, z64)
    p3 = _tc_call(_tcc_body, 128,
                  [_rows((RB, 64)), _rows((RB, 64)), _rows((RB, 64)),
                   _full((64, 128)), _full((1, 128))] + degspec)(
                      a0, a1, p2, W2, b2.reshape(1, 128), d0, d1)

    a0, a1 = _make_edge_scatter(128)(p3, srcp, dstp, z128)
    h3 = _tc_call(_tcd_body, 256,
                  [_rows((RB, 128)), _rows((RB, 128)), _rows((RB, 128)),
                   _full((128, 256)), _full((1, 256))] + degspec)(
                      a0, a1, p3, W3, b3.reshape(1, 256), d0, d1)

    psum, cnt = pl.pallas_call(
        _pool_body,
        grid=(_GRID,),
        in_specs=[_rows((RB, 1)), _rows((RB, 256))],
        out_specs=(_full((B, 256)), _full((B, 1))),
        out_shape=(jax.ShapeDtypeStruct((B, 256), _F32),
                   jax.ShapeDtypeStruct((B, 1), _F32)),
    )(batch.reshape(N, 1), h3)

    out = pl.pallas_call(
        _head_body,
        out_shape=jax.ShapeDtypeStruct((B, 1), _F32),
    )(psum, cnt, cell_features, Wd, bd.reshape(1, 64), Wc1,
      bc1.reshape(1, 128), Wc2, bc2.reshape(1, 64), Wm1,
      bm1.reshape(1, 64), Wm2, bm2.reshape(1, 32), Wo, bo.reshape(1, 1))

    return out.reshape(-1)


# speedup vs baseline: 29.8098x; 2.3990x over previous
import functools

import jax
import jax.numpy as jnp
from jax import lax
from jax.experimental import pallas as pl
import jax.experimental.pallas.tpu as pltpu
from jax.experimental.pallas import tpu_sc as plsc

N = 10000
E = 320000
B = 256
NTILES = 32
EP = E // NTILES
K = 80
G = EP // K
NACC = 10016
ZR = NACC // 16
DEGW = 8
RB = 1000
_F32 = jnp.float32
_HI = jax.lax.Precision.DEFAULT

def _dot(a, b):
    return jax.lax.dot_general(a, b, (((1,), (0,)), ((), ())),
                               precision=_HI, preferred_element_type=_F32)


@functools.lru_cache(maxsize=None)
def _make_edge_scatter(F):
    _mesh = plsc.VectorSubcoreMesh(core_axis_name="c", subcore_axis_name="s")

    @functools.partial(
        pl.kernel,
        out_type=(jax.ShapeDtypeStruct((NACC, F), _F32),
                  jax.ShapeDtypeStruct((NACC, F), _F32)),
        mesh=_mesh,
        scratch_types=[
            pltpu.VMEM((G, K), jnp.int32),
            pltpu.VMEM((G, K), jnp.int32),
            pltpu.VMEM((K, F), _F32),
            pltpu.VMEM((K, F), _F32),
            pltpu.VMEM_SHARED((NACC, F), _F32),
            pltpu.SemaphoreType.DMA,
            pltpu.SemaphoreType.DMA,
        ],
        compiler_params=pltpu.CompilerParams(use_tc_tiling_on_sc=False),
    )
    def body(p_hbm, src_hbm, dst_hbm, zer_hbm, out0, out1,
             src_all, dst_all, rows0, rows1, acc_sh, sem0, sem1):
        c = lax.axis_index("c")
        s = lax.axis_index("s")
        rows = pl.ds(s * ZR, ZR)
        pltpu.sync_copy(zer_hbm, acc_sh.at[rows])
        wid = c * 16 + s
        pltpu.sync_copy(src_hbm.at[pl.ds(wid * G, G)], src_all)
        pltpu.sync_copy(dst_hbm.at[pl.ds(wid * G, G)], dst_all)
        plsc.subcore_barrier()

        pltpu.async_copy(p_hbm.at[src_all.at[0]], rows0, sem0)
        pltpu.async_copy(p_hbm.at[src_all.at[1]], rows1, sem1)

        def pair(j, carry):
            g0 = j * 2
            pltpu.make_async_copy(p_hbm.at[src_all.at[g0]], rows0, sem0).wait()
            pltpu.sync_copy(rows0, acc_sh.at[dst_all.at[g0]], add=True)

            @pl.when(g0 + 2 < G)
            def _():
                pltpu.async_copy(p_hbm.at[src_all.at[g0 + 2]], rows0, sem0)

            pltpu.make_async_copy(p_hbm.at[src_all.at[g0 + 1]], rows1,
                                  sem1).wait()
            pltpu.sync_copy(rows1, acc_sh.at[dst_all.at[g0 + 1]], add=True)

            @pl.when(g0 + 3 < G)
            def _():
                pltpu.async_copy(p_hbm.at[src_all.at[g0 + 3]], rows1, sem1)

            return carry

        lax.fori_loop(0, G // 2, pair, 0)
        if G % 2:
            pltpu.make_async_copy(p_hbm.at[src_all.at[G - 1]], rows0,
                                  sem0).wait()
            pltpu.sync_copy(rows0, acc_sh.at[dst_all.at[G - 1]], add=True)
        plsc.subcore_barrier()

        @pl.when(c == 0)
        def _():
            pltpu.sync_copy(acc_sh.at[rows], out0.at[rows])

        @pl.when(c == 1)
        def _():
            pltpu.sync_copy(acc_sh.at[rows], out1.at[rows])

    return body


@functools.lru_cache(maxsize=None)
def _make_deg_scatter():
    _mesh = plsc.VectorSubcoreMesh(core_axis_name="c", subcore_axis_name="s")

    @functools.partial(
        pl.kernel,
        out_type=(jax.ShapeDtypeStruct((NACC, DEGW), _F32),
                  jax.ShapeDtypeStruct((NACC, DEGW), _F32)),
        mesh=_mesh,
        scratch_types=[
            pltpu.VMEM((G, K), jnp.int32),
            pltpu.VMEM((K, DEGW), _F32),
            pltpu.VMEM_SHARED((NACC, DEGW), _F32),
        ],
        compiler_params=pltpu.CompilerParams(use_tc_tiling_on_sc=False),
    )
    def body(dst_hbm, ones_hbm, zer_hbm, out0, out1, dst_all, ones_v, acc_sh):
        c = lax.axis_index("c")
        s = lax.axis_index("s")
        rows = pl.ds(s * ZR, ZR)
        pltpu.sync_copy(zer_hbm, acc_sh.at[rows])
        pltpu.sync_copy(ones_hbm, ones_v)
        wid = c * 16 + s
        pltpu.sync_copy(dst_hbm.at[pl.ds(wid * G, G)], dst_all)
        plsc.subcore_barrier()

        def step(g, carry):
            pltpu.sync_copy(ones_v, acc_sh.at[dst_all.at[g]], add=True)
            return carry

        lax.fori_loop(0, G, step, 0)
        plsc.subcore_barrier()

        @pl.when(c == 0)
        def _():
            pltpu.sync_copy(acc_sh.at[rows], out0.at[rows])

        @pl.when(c == 1)
        def _():
            pltpu.sync_copy(acc_sh.at[rows], out1.at[rows])

    return body


def _dinv_of(d0, d1):
    return 1.0 / jnp.sqrt(d0[:, :1] + d1[:, :1] + 1.0)


def _tca_body(x_ref, w1_ref, d0_ref, d1_ref, p1_ref):
    di = _dinv_of(d0_ref[...], d1_ref[...])
    p1_ref[...] = di * _dot(x_ref[...], w1_ref[...])


def _tcb_body(a0_ref, a1_ref, p1_ref, b1_ref, d0_ref, d1_ref, p2_ref):
    di = _dinv_of(d0_ref[...], d1_ref[...])
    conv = di * (a0_ref[...] + a1_ref[...] + p1_ref[...]) + b1_ref[...]
    p2_ref[...] = di * jnp.maximum(conv, 0.0)


def _tcc_body(a0_ref, a1_ref, p2_ref, w2_ref, b2_ref, d0_ref, d1_ref, p3_ref):
    di = _dinv_of(d0_ref[...], d1_ref[...])
    m = di * (a0_ref[...] + a1_ref[...] + p2_ref[...])
    conv = _dot(m, w2_ref[...]) + b2_ref[...]
    p3_ref[...] = di * jnp.maximum(conv, 0.0)


def _tcd_body(a0_ref, a1_ref, p3_ref, w3_ref, b3_ref, d0_ref, d1_ref, h3_ref):
    di = _dinv_of(d0_ref[...], d1_ref[...])
    m = di * (a0_ref[...] + a1_ref[...] + p3_ref[...])
    h3_ref[...] = jnp.maximum(_dot(m, w3_ref[...]) + b3_ref[...], 0.0)


def _pool_body(batch_ref, h3_ref, psum_ref, cnt_ref):
    @pl.when(pl.program_id(0) == 0)
    def _():
        psum_ref[...] = jnp.zeros_like(psum_ref)
        cnt_ref[...] = jnp.zeros_like(cnt_ref)

    oh = (batch_ref[...] == jax.lax.broadcasted_iota(jnp.int32, (1, B), 1))
    oh = oh.astype(_F32)
    tdot = lambda a, b: jax.lax.dot_general(
        a, b, (((0,), (0,)), ((), ())), precision=_HI,
        preferred_element_type=_F32)
    psum_ref[...] += tdot(oh, h3_ref[...])
    cnt_ref[...] += tdot(oh, jnp.ones((RB, 1), _F32))


def _head_body(psum_ref, cnt_ref, cf_ref, wd_ref, bd_ref, wc1_ref, bc1_ref,
               wc2_ref, bc2_ref, wm1_ref, bm1_ref, wm2_ref, bm2_ref,
               wo_ref, bo_ref, out_ref):
    mean = psum_ref[...] / jnp.maximum(cnt_ref[...], 1.0)
    drug = _dot(mean, wd_ref[...]) + bd_ref[...]
    cellh = jnp.maximum(_dot(cf_ref[...], wc1_ref[...]) + bc1_ref[...], 0.0)
    cell = _dot(cellh, wc2_ref[...]) + bc2_ref[...]
    wm1 = wm1_ref[...]
    z = jnp.maximum(_dot(drug, wm1[:64]) + _dot(cell, wm1[64:])
                    + bm1_ref[...], 0.0)
    z = jnp.maximum(_dot(z, wm2_ref[...]) + bm2_ref[...], 0.0)
    out_ref[...] = _dot(z, wo_ref[...]) + bo_ref[...]


def _rows(shape):
    return pl.BlockSpec(shape, lambda i: (i, 0))


def _full(shape):
    return pl.BlockSpec(shape, lambda i: (0, 0))


_GRID = N // RB


def _tc_call(body, n_out_cols, in_specs):
    return pl.pallas_call(
        body,
        grid=(_GRID,),
        in_specs=in_specs,
        out_specs=_rows((RB, n_out_cols)),
        out_shape=jax.ShapeDtypeStruct((N, n_out_cols), _F32),
    )


def kernel(x, edge_index, batch, cell_features, W1, b1, W2, b2, W3, b3,
           Wd, bd, Wc1, bc1, Wc2, bc2, Wm1, bm1, Wm2, bm2, Wo, bo):
    src = edge_index[0].astype(jnp.int32)
    dst = edge_index[1].astype(jnp.int32)
    batch = batch.astype(jnp.int32)

    srcp = src.reshape(NTILES * G, K)
    dstp = dst.reshape(NTILES * G, K)

    z16 = jnp.zeros((ZR, DEGW), _F32)
    z64 = jnp.zeros((ZR, 64), _F32)
    z128 = jnp.zeros((ZR, 128), _F32)
    ones16 = jnp.ones((K, DEGW), _F32)

    d0, d1 = _make_deg_scatter()(dstp, ones16, z16)

    degspec = [_rows((RB, DEGW)), _rows((RB, DEGW))]
    p1 = _tc_call(_tca_body, 64,
                  [_rows((RB, 128)), _full((128, 64))] + degspec)(
                      x, W1, d0, d1)

    a0, a1 = _make_edge_scatter(64)(p1, srcp, dstp, z64)
    p2 = _tc_call(_tcb_body, 64,
                  [_rows((RB, 64)), _rows((RB, 64)), _rows((RB, 64)),
                   _full((1, 64))] + degspec)(
                      a0, a1, p1, b1.reshape(1, 64), d0, d1)

    a0, a1 = _make_edge_scatter(64)(p2, srcp, dstp, z64)
    p3 = _tc_call(_tcc_body, 128,
                  [_rows((RB, 64)), _rows((RB, 64)), _rows((RB, 64)),
                   _full((64, 128)), _full((1, 128))] + degspec)(
                      a0, a1, p2, W2, b2.reshape(1, 128), d0, d1)

    a0, a1 = _make_edge_scatter(128)(p3, srcp, dstp, z128)
    h3 = _tc_call(_tcd_body, 256,
                  [_rows((RB, 128)), _rows((RB, 128)), _rows((RB, 128)),
                   _full((128, 256)), _full((1, 256))] + degspec)(
                      a0, a1, p3, W3, b3.reshape(1, 256), d0, d1)

    psum, cnt = pl.pallas_call(
        _pool_body,
        grid=(_GRID,),
        in_specs=[_rows((RB, 1)), _rows((RB, 256))],
        out_specs=(_full((B, 256)), _full((B, 1))),
        out_shape=(jax.ShapeDtypeStruct((B, 256), _F32),
                   jax.ShapeDtypeStruct((B, 1), _F32)),
    )(batch.reshape(N, 1), h3)

    out = pl.pallas_call(
        _head_body,
        out_shape=jax.ShapeDtypeStruct((B, 1), _F32),
    )(psum, cnt, cell_features, Wd, bd.reshape(1, 64), Wc1,
      bc1.reshape(1, 128), Wc2, bc2.reshape(1, 64), Wm1,
      bm1.reshape(1, 64), Wm2, bm2.reshape(1, 32), Wo, bo.reshape(1, 1))

    return out.reshape(-1)


# speedup vs baseline: 30.7381x; 1.0311x over previous
import functools

import jax
import jax.numpy as jnp
from jax import lax
from jax.experimental import pallas as pl
import jax.experimental.pallas.tpu as pltpu
from jax.experimental.pallas import tpu_sc as plsc

N = 10000
E = 320000
B = 256
NTILES = 32
EP = E // NTILES
K = 80
G = EP // K
NACC = 10016
ZR = NACC // 16
DEGW = 8
RB = 1000
_F32 = jnp.float32
_HI = jax.lax.Precision.DEFAULT

def _dot(a, b):
    return jax.lax.dot_general(a, b, (((1,), (0,)), ((), ())),
                               precision=_HI, preferred_element_type=_F32)


@functools.lru_cache(maxsize=None)
def _make_edge_scatter(F):
    _mesh = plsc.VectorSubcoreMesh(core_axis_name="c", subcore_axis_name="s")

    @functools.partial(
        pl.kernel,
        out_type=(jax.ShapeDtypeStruct((NACC, F), _F32),
                  jax.ShapeDtypeStruct((NACC, F), _F32)),
        mesh=_mesh,
        scratch_types=[
            pltpu.VMEM((G, K), jnp.int32),
            pltpu.VMEM((G, K), jnp.int32),
            pltpu.VMEM((K, F), _F32),
            pltpu.VMEM((K, F), _F32),
            pltpu.VMEM_SHARED((NACC, F), _F32),
            pltpu.SemaphoreType.DMA,
            pltpu.SemaphoreType.DMA,
        ],
        compiler_params=pltpu.CompilerParams(use_tc_tiling_on_sc=False),
    )
    def body(p_hbm, src_hbm, dst_hbm, zer_hbm, out0, out1,
             src_all, dst_all, rows0, rows1, acc_sh, sem0, sem1):
        c = lax.axis_index("c")
        s = lax.axis_index("s")
        rows = pl.ds(s * ZR, ZR)
        pltpu.sync_copy(zer_hbm, acc_sh.at[rows])
        wid = c * 16 + s
        pltpu.sync_copy(src_hbm.at[pl.ds(wid * G, G)], src_all)
        pltpu.sync_copy(dst_hbm.at[pl.ds(wid * G, G)], dst_all)
        plsc.subcore_barrier()

        pltpu.async_copy(p_hbm.at[src_all.at[0]], rows0, sem0)
        pltpu.async_copy(p_hbm.at[src_all.at[1]], rows1, sem1)

        def pair(j, carry):
            g0 = j * 2
            pltpu.make_async_copy(p_hbm.at[src_all.at[g0]], rows0, sem0).wait()
            pltpu.sync_copy(rows0, acc_sh.at[dst_all.at[g0]], add=True)

            @pl.when(g0 + 2 < G)
            def _():
                pltpu.async_copy(p_hbm.at[src_all.at[g0 + 2]], rows0, sem0)

            pltpu.make_async_copy(p_hbm.at[src_all.at[g0 + 1]], rows1,
                                  sem1).wait()
            pltpu.sync_copy(rows1, acc_sh.at[dst_all.at[g0 + 1]], add=True)

            @pl.when(g0 + 3 < G)
            def _():
                pltpu.async_copy(p_hbm.at[src_all.at[g0 + 3]], rows1, sem1)

            return carry

        lax.fori_loop(0, G // 2, pair, 0)
        if G % 2:
            pltpu.make_async_copy(p_hbm.at[src_all.at[G - 1]], rows0,
                                  sem0).wait()
            pltpu.sync_copy(rows0, acc_sh.at[dst_all.at[G - 1]], add=True)
        plsc.subcore_barrier()

        @pl.when(c == 0)
        def _():
            pltpu.sync_copy(acc_sh.at[rows], out0.at[rows])

        @pl.when(c == 1)
        def _():
            pltpu.sync_copy(acc_sh.at[rows], out1.at[rows])

    return body


@functools.lru_cache(maxsize=None)
def _make_deg_scatter():
    _mesh = plsc.VectorSubcoreMesh(core_axis_name="c", subcore_axis_name="s")

    @functools.partial(
        pl.kernel,
        out_type=(jax.ShapeDtypeStruct((NACC, DEGW), _F32),
                  jax.ShapeDtypeStruct((NACC, DEGW), _F32)),
        mesh=_mesh,
        scratch_types=[
            pltpu.VMEM((G, K), jnp.int32),
            pltpu.VMEM((K, DEGW), _F32),
            pltpu.VMEM_SHARED((NACC, DEGW), _F32),
        ],
        compiler_params=pltpu.CompilerParams(use_tc_tiling_on_sc=False),
    )
    def body(dst_hbm, ones_hbm, zer_hbm, out0, out1, dst_all, ones_v, acc_sh):
        c = lax.axis_index("c")
        s = lax.axis_index("s")
        rows = pl.ds(s * ZR, ZR)
        pltpu.sync_copy(zer_hbm, acc_sh.at[rows])
        pltpu.sync_copy(ones_hbm, ones_v)
        wid = c * 16 + s
        pltpu.sync_copy(dst_hbm.at[pl.ds(wid * G, G)], dst_all)
        plsc.subcore_barrier()

        def step(g, carry):
            pltpu.sync_copy(ones_v, acc_sh.at[dst_all.at[g]], add=True)
            return carry

        lax.fori_loop(0, G, step, 0)
        plsc.subcore_barrier()

        @pl.when(c == 0)
        def _():
            pltpu.sync_copy(acc_sh.at[rows], out0.at[rows])

        @pl.when(c == 1)
        def _():
            pltpu.sync_copy(acc_sh.at[rows], out1.at[rows])

    return body


def _dinv_of(d0, d1):
    return 1.0 / jnp.sqrt(d0[:, :1] + d1[:, :1] + 1.0)


def _tca_body(x_ref, w1_ref, d0_ref, d1_ref, p1_ref):
    di = _dinv_of(d0_ref[...], d1_ref[...])
    p1_ref[...] = di * _dot(x_ref[...], w1_ref[...])


def _tcb_body(a0_ref, a1_ref, p1_ref, b1_ref, d0_ref, d1_ref, p2_ref):
    di = _dinv_of(d0_ref[...], d1_ref[...])
    conv = di * (a0_ref[...] + a1_ref[...] + p1_ref[...]) + b1_ref[...]
    p2_ref[...] = di * jnp.maximum(conv, 0.0)


def _tcc_body(a0_ref, a1_ref, p2_ref, w2_ref, b2_ref, d0_ref, d1_ref, p3_ref):
    di = _dinv_of(d0_ref[...], d1_ref[...])
    m = di * (a0_ref[...] + a1_ref[...] + p2_ref[...])
    conv = _dot(m, w2_ref[...]) + b2_ref[...]
    p3_ref[...] = di * jnp.maximum(conv, 0.0)


def _tcd_body(a0_ref, a1_ref, p3_ref, w3_ref, b3_ref, d0_ref, d1_ref,
              batch_ref, cf_ref, wd_ref, bd_ref, wc1_ref, bc1_ref,
              wc2_ref, bc2_ref, wm1_ref, bm1_ref, wm2_ref, bm2_ref,
              wo_ref, bo_ref, out_ref, psum_s, cnt_s):
    i = pl.program_id(0)

    @pl.when(i == 0)
    def _():
        psum_s[...] = jnp.zeros_like(psum_s)
        cnt_s[...] = jnp.zeros_like(cnt_s)

    di = _dinv_of(d0_ref[...], d1_ref[...])
    m = di * (a0_ref[...] + a1_ref[...] + p3_ref[...])
    h3 = jnp.maximum(_dot(m, w3_ref[...]) + b3_ref[...], 0.0)
    oh = (batch_ref[...] == jax.lax.broadcasted_iota(jnp.int32, (1, B), 1))
    oh = oh.astype(_F32)
    tdot = lambda a, b: jax.lax.dot_general(
        a, b, (((0,), (0,)), ((), ())), precision=_HI,
        preferred_element_type=_F32)
    psum_s[...] += tdot(oh, h3)
    cnt_s[...] += tdot(oh, jnp.ones((RB, 1), _F32))

    @pl.when(i == _GRID - 1)
    def _():
        mean = psum_s[...] / jnp.maximum(cnt_s[...], 1.0)
        drug = _dot(mean, wd_ref[...]) + bd_ref[...]
        cellh = jnp.maximum(_dot(cf_ref[...], wc1_ref[...]) + bc1_ref[...],
                            0.0)
        cell = _dot(cellh, wc2_ref[...]) + bc2_ref[...]
        wm1 = wm1_ref[...]
        z = jnp.maximum(_dot(drug, wm1[:64]) + _dot(cell, wm1[64:])
                        + bm1_ref[...], 0.0)
        z = jnp.maximum(_dot(z, wm2_ref[...]) + bm2_ref[...], 0.0)
        out_ref[...] = _dot(z, wo_ref[...]) + bo_ref[...]


def _rows(shape):
    return pl.BlockSpec(shape, lambda i: (i, 0))


def _full(shape):
    return pl.BlockSpec(shape, lambda i: (0, 0))


_GRID = N // RB


def _tc_call(body, n_out_cols, in_specs):
    return pl.pallas_call(
        body,
        grid=(_GRID,),
        in_specs=in_specs,
        out_specs=_rows((RB, n_out_cols)),
        out_shape=jax.ShapeDtypeStruct((N, n_out_cols), _F32),
    )


def kernel(x, edge_index, batch, cell_features, W1, b1, W2, b2, W3, b3,
           Wd, bd, Wc1, bc1, Wc2, bc2, Wm1, bm1, Wm2, bm2, Wo, bo):
    src = edge_index[0].astype(jnp.int32)
    dst = edge_index[1].astype(jnp.int32)
    batch = batch.astype(jnp.int32)

    srcp = src.reshape(NTILES * G, K)
    dstp = dst.reshape(NTILES * G, K)

    z16 = jnp.zeros((ZR, DEGW), _F32)
    z64 = jnp.zeros((ZR, 64), _F32)
    z128 = jnp.zeros((ZR, 128), _F32)
    ones16 = jnp.ones((K, DEGW), _F32)

    d0, d1 = _make_deg_scatter()(dstp, ones16, z16)

    degspec = [_rows((RB, DEGW)), _rows((RB, DEGW))]
    p1 = _tc_call(_tca_body, 64,
                  [_rows((RB, 128)), _full((128, 64))] + degspec)(
                      x, W1, d0, d1)

    a0, a1 = _make_edge_scatter(64)(p1, srcp, dstp, z64)
    p2 = _tc_call(_tcb_body, 64,
                  [_rows((RB, 64)), _rows((RB, 64)), _rows((RB, 64)),
                   _full((1, 64))] + degspec)(
                      a0, a1, p1, b1.reshape(1, 64), d0, d1)

    a0, a1 = _make_edge_scatter(64)(p2, srcp, dstp, z64)
    p3 = _tc_call(_tcc_body, 128,
                  [_rows((RB, 64)), _rows((RB, 64)), _rows((RB, 64)),
                   _full((64, 128)), _full((1, 128))] + degspec)(
                      a0, a1, p2, W2, b2.reshape(1, 128), d0, d1)

    a0, a1 = _make_edge_scatter(128)(p3, srcp, dstp, z128)
    out = pl.pallas_call(
        _tcd_body,
        grid=(_GRID,),
        in_specs=[_rows((RB, 128)), _rows((RB, 128)), _rows((RB, 128)),
                  _full((128, 256)), _full((1, 256))] + degspec +
                 [_rows((RB, 1)), _full((B, 512)), _full((256, 64)),
                  _full((1, 64)), _full((512, 128)), _full((1, 128)),
                  _full((128, 64)), _full((1, 64)), _full((128, 64)),
                  _full((1, 64)), _full((64, 32)), _full((1, 32)),
                  _full((32, 1)), _full((1, 1))],
        out_specs=_full((B, 1)),
        out_shape=jax.ShapeDtypeStruct((B, 1), _F32),
        scratch_shapes=[pltpu.VMEM((B, 256), _F32), pltpu.VMEM((B, 1), _F32)],
    )(a0, a1, p3, W3, b3.reshape(1, 256), d0, d1, batch.reshape(N, 1),
      cell_features, Wd, bd.reshape(1, 64), Wc1, bc1.reshape(1, 128),
      Wc2, bc2.reshape(1, 64), Wm1, bm1.reshape(1, 64), Wm2,
      bm2.reshape(1, 32), Wo, bo.reshape(1, 1))

    return out.reshape(-1)


# speedup vs baseline: 33.1409x; 1.0782x over previous
import functools

import jax
import jax.numpy as jnp
from jax import lax
from jax.experimental import pallas as pl
import jax.experimental.pallas.tpu as pltpu
from jax.experimental.pallas import tpu_sc as plsc

N = 10000
E = 320000
B = 256
NTILES = 32
EP = E // NTILES
K = 80
G = EP // K
NACC = 10000
ZR = NACC // 16
DEGW = 8
RB = 1000
_F32 = jnp.float32
_HI = jax.lax.Precision.DEFAULT

def _dot(a, b):
    return jax.lax.dot_general(a, b, (((1,), (0,)), ((), ())),
                               precision=_HI, preferred_element_type=_F32)


@functools.lru_cache(maxsize=None)
def _make_edge_scatter(F):
    NBUF = 4 if F <= 64 else 3
    _mesh = plsc.VectorSubcoreMesh(core_axis_name="c", subcore_axis_name="s")

    @functools.partial(
        pl.kernel,
        out_type=(jax.ShapeDtypeStruct((NACC, F), _F32),
                  jax.ShapeDtypeStruct((NACC, F), _F32)),
        mesh=_mesh,
        scratch_types=(
            [pltpu.VMEM((G, K), jnp.int32), pltpu.VMEM((G, K), jnp.int32)]
            + [pltpu.VMEM((K, F), _F32) for _ in range(NBUF)]
            + [pltpu.VMEM_SHARED((NACC, F), _F32)]
            + [pltpu.SemaphoreType.DMA for _ in range(2 * NBUF)]
        ),
        compiler_params=pltpu.CompilerParams(use_tc_tiling_on_sc=False),
    )
    def body(p_hbm, src_hbm, dst_hbm, zer_hbm, out0, out1,
             src_all, dst_all, *bufs_acc_sems):
        rows = bufs_acc_sems[:NBUF]
        acc_sh = bufs_acc_sems[NBUF]
        gsem = bufs_acc_sems[NBUF + 1:2 * NBUF + 1]
        ssem = bufs_acc_sems[2 * NBUF + 1:]
        c = lax.axis_index("c")
        s = lax.axis_index("s")
        myrows = pl.ds(s * ZR, ZR)
        pltpu.sync_copy(zer_hbm, acc_sh.at[myrows])
        wid = c * 16 + s
        pltpu.sync_copy(src_hbm.at[pl.ds(wid * G, G)], src_all)
        pltpu.sync_copy(dst_hbm.at[pl.ds(wid * G, G)], dst_all)
        plsc.subcore_barrier()

        for b in range(NBUF):
            pltpu.async_copy(p_hbm.at[src_all.at[b]], rows[b], gsem[b])

        def rnd(r, carry):
            g0 = r * NBUF
            for b in range(NBUF):
                pltpu.make_async_copy(p_hbm.at[src_all.at[g0 + b]], rows[b],
                                      gsem[b]).wait()
                pltpu.async_copy(rows[b], acc_sh.at[dst_all.at[g0 + b]],
                                 ssem[b], add=True)
            for b in range(NBUF):
                pltpu.make_async_copy(rows[b], acc_sh.at[dst_all.at[g0 + b]],
                                      ssem[b]).wait()

                @pl.when(g0 + NBUF + b < G)
                def _():
                    pltpu.async_copy(p_hbm.at[src_all.at[g0 + NBUF + b]],
                                     rows[b], gsem[b])

            return carry

        lax.fori_loop(0, G // NBUF, rnd, 0)
        for b in range(G % NBUF):
            g = (G // NBUF) * NBUF + b
            pltpu.make_async_copy(p_hbm.at[src_all.at[g]], rows[b],
                                  gsem[b]).wait()
            pltpu.sync_copy(rows[b], acc_sh.at[dst_all.at[g]], add=True)
        plsc.subcore_barrier()

        @pl.when(c == 0)
        def _():
            pltpu.sync_copy(acc_sh.at[myrows], out0.at[myrows])

        @pl.when(c == 1)
        def _():
            pltpu.sync_copy(acc_sh.at[myrows], out1.at[myrows])

    return body


@functools.lru_cache(maxsize=None)
def _make_deg_scatter():
    _mesh = plsc.VectorSubcoreMesh(core_axis_name="c", subcore_axis_name="s")

    @functools.partial(
        pl.kernel,
        out_type=(jax.ShapeDtypeStruct((NACC, DEGW), _F32),
                  jax.ShapeDtypeStruct((NACC, DEGW), _F32)),
        mesh=_mesh,
        scratch_types=[
            pltpu.VMEM((G, K), jnp.int32),
            pltpu.VMEM((K, DEGW), _F32),
            pltpu.VMEM_SHARED((NACC, DEGW), _F32),
        ],
        compiler_params=pltpu.CompilerParams(use_tc_tiling_on_sc=False),
    )
    def body(dst_hbm, ones_hbm, zer_hbm, out0, out1, dst_all, ones_v, acc_sh):
        c = lax.axis_index("c")
        s = lax.axis_index("s")
        rows = pl.ds(s * ZR, ZR)
        pltpu.sync_copy(zer_hbm, acc_sh.at[rows])
        pltpu.sync_copy(ones_hbm, ones_v)
        wid = c * 16 + s
        pltpu.sync_copy(dst_hbm.at[pl.ds(wid * G, G)], dst_all)
        plsc.subcore_barrier()

        def step(g, carry):
            pltpu.sync_copy(ones_v, acc_sh.at[dst_all.at[g]], add=True)
            return carry

        lax.fori_loop(0, G, step, 0)
        plsc.subcore_barrier()

        @pl.when(c == 0)
        def _():
            pltpu.sync_copy(acc_sh.at[rows], out0.at[rows])

        @pl.when(c == 1)
        def _():
            pltpu.sync_copy(acc_sh.at[rows], out1.at[rows])

    return body


def _dinv_of(d0, d1):
    return 1.0 / jnp.sqrt(d0[:, :1] + d1[:, :1] + 1.0)


def _tca_body(x_ref, w1_ref, d0_ref, d1_ref, p1_ref):
    di = _dinv_of(d0_ref[...], d1_ref[...])
    p1_ref[...] = di * _dot(x_ref[...], w1_ref[...])


def _tcb_body(a0_ref, a1_ref, p1_ref, b1_ref, d0_ref, d1_ref, p2_ref):
    di = _dinv_of(d0_ref[...], d1_ref[...])
    conv = di * (a0_ref[...] + a1_ref[...] + p1_ref[...]) + b1_ref[...]
    p2_ref[...] = di * jnp.maximum(conv, 0.0)


def _tcc_body(a0_ref, a1_ref, p2_ref, w2_ref, b2_ref, d0_ref, d1_ref, p3_ref):
    di = _dinv_of(d0_ref[...], d1_ref[...])
    m = di * (a0_ref[...] + a1_ref[...] + p2_ref[...])
    conv = _dot(m, w2_ref[...]) + b2_ref[...]
    p3_ref[...] = di * jnp.maximum(conv, 0.0)


def _tcd_body(a0_ref, a1_ref, p3_ref, w3_ref, b3_ref, d0_ref, d1_ref,
              batch_ref, cf_ref, wd_ref, bd_ref, wc1_ref, bc1_ref,
              wc2_ref, bc2_ref, wm1_ref, bm1_ref, wm2_ref, bm2_ref,
              wo_ref, bo_ref, out_ref, psum_s, cnt_s):
    i = pl.program_id(0)

    @pl.when(i == 0)
    def _():
        psum_s[...] = jnp.zeros_like(psum_s)
        cnt_s[...] = jnp.zeros_like(cnt_s)

    di = _dinv_of(d0_ref[...], d1_ref[...])
    m = di * (a0_ref[...] + a1_ref[...] + p3_ref[...])
    h3 = jnp.maximum(_dot(m, w3_ref[...]) + b3_ref[...], 0.0)
    oh = (batch_ref[...] == jax.lax.broadcasted_iota(jnp.int32, (1, B), 1))
    oh = oh.astype(_F32)
    tdot = lambda a, b: jax.lax.dot_general(
        a, b, (((0,), (0,)), ((), ())), precision=_HI,
        preferred_element_type=_F32)
    psum_s[...] += tdot(oh, h3)
    cnt_s[...] += tdot(oh, jnp.ones((RB, 1), _F32))

    @pl.when(i == _GRID - 1)
    def _():
        mean = psum_s[...] / jnp.maximum(cnt_s[...], 1.0)
        drug = _dot(mean, wd_ref[...]) + bd_ref[...]
        cellh = jnp.maximum(_dot(cf_ref[...], wc1_ref[...]) + bc1_ref[...],
                            0.0)
        cell = _dot(cellh, wc2_ref[...]) + bc2_ref[...]
        wm1 = wm1_ref[...]
        z = jnp.maximum(_dot(drug, wm1[:64]) + _dot(cell, wm1[64:])
                        + bm1_ref[...], 0.0)
        z = jnp.maximum(_dot(z, wm2_ref[...]) + bm2_ref[...], 0.0)
        out_ref[...] = _dot(z, wo_ref[...]) + bo_ref[...]


def _rows(shape):
    return pl.BlockSpec(shape, lambda i: (i, 0))


def _full(shape):
    return pl.BlockSpec(shape, lambda i: (0, 0))


_GRID = N // RB


def _tc_call(body, n_out_cols, in_specs):
    return pl.pallas_call(
        body,
        grid=(_GRID,),
        in_specs=in_specs,
        out_specs=_rows((RB, n_out_cols)),
        out_shape=jax.ShapeDtypeStruct((N, n_out_cols), _F32),
    )


def kernel(x, edge_index, batch, cell_features, W1, b1, W2, b2, W3, b3,
           Wd, bd, Wc1, bc1, Wc2, bc2, Wm1, bm1, Wm2, bm2, Wo, bo):
    src = edge_index[0].astype(jnp.int32)
    dst = edge_index[1].astype(jnp.int32)
    batch = batch.astype(jnp.int32)

    srcp = src.reshape(NTILES * G, K)
    dstp = dst.reshape(NTILES * G, K)

    z16 = jnp.zeros((ZR, DEGW), _F32)
    z64 = jnp.zeros((ZR, 64), _F32)
    z128 = jnp.zeros((ZR, 128), _F32)
    ones16 = jnp.ones((K, DEGW), _F32)

    d0, d1 = _make_deg_scatter()(dstp, ones16, z16)

    degspec = [_rows((RB, DEGW)), _rows((RB, DEGW))]
    p1 = _tc_call(_tca_body, 64,
                  [_rows((RB, 128)), _full((128, 64))] + degspec)(
                      x, W1, d0, d1)

    a0, a1 = _make_edge_scatter(64)(p1, srcp, dstp, z64)
    p2 = _tc_call(_tcb_body, 64,
                  [_rows((RB, 64)), _rows((RB, 64)), _rows((RB, 64)),
                   _full((1, 64))] + degspec)(
                      a0, a1, p1, b1.reshape(1, 64), d0, d1)

    a0, a1 = _make_edge_scatter(64)(p2, srcp, dstp, z64)
    p3 = _tc_call(_tcc_body, 128,
                  [_rows((RB, 64)), _rows((RB, 64)), _rows((RB, 64)),
                   _full((64, 128)), _full((1, 128))] + degspec)(
                      a0, a1, p2, W2, b2.reshape(1, 128), d0, d1)

    a0, a1 = _make_edge_scatter(128)(p3, srcp, dstp, z128)
    out = pl.pallas_call(
        _tcd_body,
        grid=(_GRID,),
        in_specs=[_rows((RB, 128)), _rows((RB, 128)), _rows((RB, 128)),
                  _full((128, 256)), _full((1, 256))] + degspec +
                 [_rows((RB, 1)), _full((B, 512)), _full((256, 64)),
                  _full((1, 64)), _full((512, 128)), _full((1, 128)),
                  _full((128, 64)), _full((1, 64)), _full((128, 64)),
                  _full((1, 64)), _full((64, 32)), _full((1, 32)),
                  _full((32, 1)), _full((1, 1))],
        out_specs=_full((B, 1)),
        out_shape=jax.ShapeDtypeStruct((B, 1), _F32),
        scratch_shapes=[pltpu.VMEM((B, 256), _F32), pltpu.VMEM((B, 1), _F32)],
    )(a0, a1, p3, W3, b3.reshape(1, 256), d0, d1, batch.reshape(N, 1),
      cell_features, Wd, bd.reshape(1, 64), Wc1, bc1.reshape(1, 128),
      Wc2, bc2.reshape(1, 64), Wm1, bm1.reshape(1, 64), Wm2,
      bm2.reshape(1, 32), Wo, bo.reshape(1, 1))

    return out.reshape(-1)


# speedup vs baseline: 34.2894x; 1.0347x over previous
import functools

import jax
import jax.numpy as jnp
from jax import lax
from jax.experimental import pallas as pl
import jax.experimental.pallas.tpu as pltpu
from jax.experimental.pallas import tpu_sc as plsc

N = 10000
E = 320000
B = 256
NTILES = 32
EP = E // NTILES
K = 80
G = EP // K
NACC = 10000
ZR = NACC // 16
DEGW = 8
RB = 2000
_F32 = jnp.float32
_HI = jax.lax.Precision.DEFAULT

def _dot(a, b):
    return jax.lax.dot_general(a, b, (((1,), (0,)), ((), ())),
                               precision=_HI, preferred_element_type=_F32)


@functools.lru_cache(maxsize=None)
def _make_edge_scatter(F):
    NBUF = 4 if F <= 64 else 3
    _mesh = plsc.VectorSubcoreMesh(core_axis_name="c", subcore_axis_name="s")

    @functools.partial(
        pl.kernel,
        out_type=(jax.ShapeDtypeStruct((NACC, F), _F32),
                  jax.ShapeDtypeStruct((NACC, F), _F32)),
        mesh=_mesh,
        scratch_types=(
            [pltpu.VMEM((G, K), jnp.int32), pltpu.VMEM((G, K), jnp.int32)]
            + [pltpu.VMEM((K, F), _F32) for _ in range(NBUF)]
            + [pltpu.VMEM_SHARED((NACC, F), _F32)]
            + [pltpu.SemaphoreType.DMA for _ in range(2 * NBUF)]
        ),
        compiler_params=pltpu.CompilerParams(use_tc_tiling_on_sc=False),
    )
    def body(p_hbm, src_hbm, dst_hbm, zer_hbm, out0, out1,
             src_all, dst_all, *bufs_acc_sems):
        rows = bufs_acc_sems[:NBUF]
        acc_sh = bufs_acc_sems[NBUF]
        gsem = bufs_acc_sems[NBUF + 1:2 * NBUF + 1]
        ssem = bufs_acc_sems[2 * NBUF + 1:]
        c = lax.axis_index("c")
        s = lax.axis_index("s")
        myrows = pl.ds(s * ZR, ZR)
        pltpu.sync_copy(zer_hbm, acc_sh.at[myrows])
        wid = c * 16 + s
        pltpu.sync_copy(src_hbm.at[pl.ds(wid * G, G)], src_all)
        pltpu.sync_copy(dst_hbm.at[pl.ds(wid * G, G)], dst_all)
        plsc.subcore_barrier()

        for b in range(NBUF):
            pltpu.async_copy(p_hbm.at[src_all.at[b]], rows[b], gsem[b])

        def rnd(r, carry):
            g0 = r * NBUF
            for b in range(NBUF):
                pltpu.make_async_copy(p_hbm.at[src_all.at[g0 + b]], rows[b],
                                      gsem[b]).wait()
                pltpu.async_copy(rows[b], acc_sh.at[dst_all.at[g0 + b]],
                                 ssem[b], add=True)
            for b in range(NBUF):
                pltpu.make_async_copy(rows[b], acc_sh.at[dst_all.at[g0 + b]],
                                      ssem[b]).wait()

                @pl.when(g0 + NBUF + b < G)
                def _():
                    pltpu.async_copy(p_hbm.at[src_all.at[g0 + NBUF + b]],
                                     rows[b], gsem[b])

            return carry

        lax.fori_loop(0, G // NBUF, rnd, 0)
        for b in range(G % NBUF):
            g = (G // NBUF) * NBUF + b
            pltpu.make_async_copy(p_hbm.at[src_all.at[g]], rows[b],
                                  gsem[b]).wait()
            pltpu.sync_copy(rows[b], acc_sh.at[dst_all.at[g]], add=True)
        plsc.subcore_barrier()

        @pl.when(c == 0)
        def _():
            pltpu.sync_copy(acc_sh.at[myrows], out0.at[myrows])

        @pl.when(c == 1)
        def _():
            pltpu.sync_copy(acc_sh.at[myrows], out1.at[myrows])

    return body


@functools.lru_cache(maxsize=None)
def _make_deg_scatter():
    _mesh = plsc.VectorSubcoreMesh(core_axis_name="c", subcore_axis_name="s")

    @functools.partial(
        pl.kernel,
        out_type=(jax.ShapeDtypeStruct((NACC, DEGW), _F32),
                  jax.ShapeDtypeStruct((NACC, DEGW), _F32)),
        mesh=_mesh,
        scratch_types=[
            pltpu.VMEM((G, K), jnp.int32),
            pltpu.VMEM((K, DEGW), _F32),
            pltpu.VMEM_SHARED((NACC, DEGW), _F32),
            pltpu.SemaphoreType.DMA,
        ],
        compiler_params=pltpu.CompilerParams(use_tc_tiling_on_sc=False),
    )
    def body(dst_hbm, ones_hbm, zer_hbm, out0, out1, dst_all, ones_v, acc_sh,
             sem):
        c = lax.axis_index("c")
        s = lax.axis_index("s")
        rows = pl.ds(s * ZR, ZR)
        pltpu.sync_copy(zer_hbm, acc_sh.at[rows])
        pltpu.sync_copy(ones_hbm, ones_v)
        wid = c * 16 + s
        pltpu.sync_copy(dst_hbm.at[pl.ds(wid * G, G)], dst_all)
        plsc.subcore_barrier()

        W = 8

        def rnd(r, carry):
            for b in range(W):
                pltpu.async_copy(ones_v, acc_sh.at[dst_all.at[r * W + b]],
                                 sem, add=True)
            for b in range(W):
                pltpu.make_async_copy(ones_v, acc_sh.at[dst_all.at[r * W + b]],
                                      sem).wait()
            return carry

        lax.fori_loop(0, G // W, rnd, 0)
        for g in range((G // W) * W, G):
            pltpu.sync_copy(ones_v, acc_sh.at[dst_all.at[g]], add=True)
        plsc.subcore_barrier()

        @pl.when(c == 0)
        def _():
            pltpu.sync_copy(acc_sh.at[rows], out0.at[rows])

        @pl.when(c == 1)
        def _():
            pltpu.sync_copy(acc_sh.at[rows], out1.at[rows])

    return body


def _dinv_of(d0, d1):
    return 1.0 / jnp.sqrt(d0[:, :1] + d1[:, :1] + 1.0)


def _tca_body(x_ref, w1_ref, d0_ref, d1_ref, p1_ref):
    di = _dinv_of(d0_ref[...], d1_ref[...])
    p1_ref[...] = di * _dot(x_ref[...], w1_ref[...])


def _tcb_body(a0_ref, a1_ref, p1_ref, b1_ref, d0_ref, d1_ref, p2_ref):
    di = _dinv_of(d0_ref[...], d1_ref[...])
    conv = di * (a0_ref[...] + a1_ref[...] + p1_ref[...]) + b1_ref[...]
    p2_ref[...] = di * jnp.maximum(conv, 0.0)


def _tcc_body(a0_ref, a1_ref, p2_ref, w2_ref, b2_ref, d0_ref, d1_ref, p3_ref):
    di = _dinv_of(d0_ref[...], d1_ref[...])
    m = di * (a0_ref[...] + a1_ref[...] + p2_ref[...])
    conv = _dot(m, w2_ref[...]) + b2_ref[...]
    p3_ref[...] = di * jnp.maximum(conv, 0.0)


def _tcd_body(a0_ref, a1_ref, p3_ref, w3_ref, b3_ref, d0_ref, d1_ref,
              batch_ref, cf_ref, wd_ref, bd_ref, wc1_ref, bc1_ref,
              wc2_ref, bc2_ref, wm1_ref, bm1_ref, wm2_ref, bm2_ref,
              wo_ref, bo_ref, out_ref, psum_s, cnt_s):
    i = pl.program_id(0)

    @pl.when(i == 0)
    def _():
        psum_s[...] = jnp.zeros_like(psum_s)
        cnt_s[...] = jnp.zeros_like(cnt_s)

    di = _dinv_of(d0_ref[...], d1_ref[...])
    m = di * (a0_ref[...] + a1_ref[...] + p3_ref[...])
    h3 = jnp.maximum(_dot(m, w3_ref[...]) + b3_ref[...], 0.0)
    oh = (batch_ref[...] == jax.lax.broadcasted_iota(jnp.int32, (1, B), 1))
    oh = oh.astype(_F32)
    tdot = lambda a, b: jax.lax.dot_general(
        a, b, (((0,), (0,)), ((), ())), precision=_HI,
        preferred_element_type=_F32)
    psum_s[...] += tdot(oh, h3)
    cnt_s[...] += tdot(oh, jnp.ones((RB, 1), _F32))

    @pl.when(i == _GRID - 1)
    def _():
        mean = psum_s[...] / jnp.maximum(cnt_s[...], 1.0)
        drug = _dot(mean, wd_ref[...]) + bd_ref[...]
        cellh = jnp.maximum(_dot(cf_ref[...], wc1_ref[...]) + bc1_ref[...],
                            0.0)
        cell = _dot(cellh, wc2_ref[...]) + bc2_ref[...]
        wm1 = wm1_ref[...]
        z = jnp.maximum(_dot(drug, wm1[:64]) + _dot(cell, wm1[64:])
                        + bm1_ref[...], 0.0)
        z = jnp.maximum(_dot(z, wm2_ref[...]) + bm2_ref[...], 0.0)
        out_ref[...] = _dot(z, wo_ref[...]) + bo_ref[...]


def _rows(shape):
    return pl.BlockSpec(shape, lambda i: (i, 0))


def _full(shape):
    return pl.BlockSpec(shape, lambda i: (0, 0))


_GRID = N // RB


def _tc_call(body, n_out_cols, in_specs):
    return pl.pallas_call(
        body,
        grid=(_GRID,),
        in_specs=in_specs,
        out_specs=_rows((RB, n_out_cols)),
        out_shape=jax.ShapeDtypeStruct((N, n_out_cols), _F32),
    )


def kernel(x, edge_index, batch, cell_features, W1, b1, W2, b2, W3, b3,
           Wd, bd, Wc1, bc1, Wc2, bc2, Wm1, bm1, Wm2, bm2, Wo, bo):
    src = edge_index[0].astype(jnp.int32)
    dst = edge_index[1].astype(jnp.int32)
    batch = batch.astype(jnp.int32)

    srcp = src.reshape(NTILES * G, K)
    dstp = dst.reshape(NTILES * G, K)

    z16 = jnp.zeros((ZR, DEGW), _F32)
    z64 = jnp.zeros((ZR, 64), _F32)
    z128 = jnp.zeros((ZR, 128), _F32)
    ones16 = jnp.ones((K, DEGW), _F32)

    d0, d1 = _make_deg_scatter()(dstp, ones16, z16)

    degspec = [_rows((RB, DEGW)), _rows((RB, DEGW))]
    p1 = _tc_call(_tca_body, 64,
                  [_rows((RB, 128)), _full((128, 64))] + degspec)(
                      x, W1, d0, d1)

    a0, a1 = _make_edge_scatter(64)(p1, srcp, dstp, z64)
    p2 = _tc_call(_tcb_body, 64,
                  [_rows((RB, 64)), _rows((RB, 64)), _rows((RB, 64)),
                   _full((1, 64))] + degspec)(
                      a0, a1, p1, b1.reshape(1, 64), d0, d1)

    a0, a1 = _make_edge_scatter(64)(p2, srcp, dstp, z64)
    p3 = _tc_call(_tcc_body, 128,
                  [_rows((RB, 64)), _rows((RB, 64)), _rows((RB, 64)),
                   _full((64, 128)), _full((1, 128))] + degspec)(
                      a0, a1, p2, W2, b2.reshape(1, 128), d0, d1)

    a0, a1 = _make_edge_scatter(128)(p3, srcp, dstp, z128)
    out = pl.pallas_call(
        _tcd_body,
        grid=(_GRID,),
        in_specs=[_rows((RB, 128)), _rows((RB, 128)), _rows((RB, 128)),
                  _full((128, 256)), _full((1, 256))] + degspec +
                 [_rows((RB, 1)), _full((B, 512)), _full((256, 64)),
                  _full((1, 64)), _full((512, 128)), _full((1, 128)),
                  _full((128, 64)), _full((1, 64)), _full((128, 64)),
                  _full((1, 64)), _full((64, 32)), _full((1, 32)),
                  _full((32, 1)), _full((1, 1))],
        out_specs=_full((B, 1)),
        out_shape=jax.ShapeDtypeStruct((B, 1), _F32),
        scratch_shapes=[pltpu.VMEM((B, 256), _F32), pltpu.VMEM((B, 1), _F32)],
    )(a0, a1, p3, W3, b3.reshape(1, 256), d0, d1, batch.reshape(N, 1),
      cell_features, Wd, bd.reshape(1, 64), Wc1, bc1.reshape(1, 128),
      Wc2, bc2.reshape(1, 64), Wm1, bm1.reshape(1, 64), Wm2,
      bm2.reshape(1, 32), Wo, bo.reshape(1, 1))

    return out.reshape(-1)


# speedup vs baseline: 34.8326x; 1.0158x over previous
import functools

import jax
import jax.numpy as jnp
from jax import lax
from jax.experimental import pallas as pl
import jax.experimental.pallas.tpu as pltpu
from jax.experimental.pallas import tpu_sc as plsc

N = 10000
E = 320000
B = 256
NTILES = 32
EP = E // NTILES
K = 80
G = EP // K
NACC = 10000
ZR = NACC // 16
DEGW = 8
RB = 2000
_F32 = jnp.float32
_HI = jax.lax.Precision.DEFAULT

def _dot(a, b):
    return jax.lax.dot_general(a, b, (((1,), (0,)), ((), ())),
                               precision=_HI, preferred_element_type=_F32)


@functools.lru_cache(maxsize=None)
def _make_edge_scatter(F):
    NBUF = 4 if F <= 64 else 3
    _mesh = plsc.VectorSubcoreMesh(core_axis_name="c", subcore_axis_name="s")

    @functools.partial(
        pl.kernel,
        out_type=(jax.ShapeDtypeStruct((NACC, F), _F32),
                  jax.ShapeDtypeStruct((NACC, F), _F32)),
        mesh=_mesh,
        scratch_types=(
            [pltpu.VMEM((G, K), jnp.int32), pltpu.VMEM((G, K), jnp.int32)]
            + [pltpu.VMEM((K, F), _F32) for _ in range(NBUF)]
            + [pltpu.VMEM_SHARED((NACC, F), _F32)]
            + [pltpu.SemaphoreType.DMA for _ in range(2 * NBUF)]
        ),
        compiler_params=pltpu.CompilerParams(use_tc_tiling_on_sc=False),
    )
    def body(p_hbm, src_hbm, dst_hbm, zer_hbm, out0, out1,
             src_all, dst_all, *bufs_acc_sems):
        rows = bufs_acc_sems[:NBUF]
        acc_sh = bufs_acc_sems[NBUF]
        gsem = bufs_acc_sems[NBUF + 1:2 * NBUF + 1]
        ssem = bufs_acc_sems[2 * NBUF + 1:]
        c = lax.axis_index("c")
        s = lax.axis_index("s")
        myrows = pl.ds(s * ZR, ZR)
        pltpu.sync_copy(zer_hbm, rows[0])
        for z in range(ZR // K):
            pltpu.sync_copy(rows[0], acc_sh.at[pl.ds(s * ZR + z * K, K)])
        if ZR % K:
            pltpu.sync_copy(rows[0].at[pl.ds(0, ZR % K)],
                            acc_sh.at[pl.ds(s * ZR + (ZR // K) * K, ZR % K)])
        wid = c * 16 + s
        pltpu.sync_copy(src_hbm.at[pl.ds(wid * G, G)], src_all)
        pltpu.sync_copy(dst_hbm.at[pl.ds(wid * G, G)], dst_all)
        plsc.subcore_barrier()

        for b in range(NBUF):
            pltpu.async_copy(p_hbm.at[src_all.at[b]], rows[b], gsem[b])

        def rnd(r, carry):
            g0 = r * NBUF
            for b in range(NBUF):
                pltpu.make_async_copy(p_hbm.at[src_all.at[g0 + b]], rows[b],
                                      gsem[b]).wait()
                pltpu.async_copy(rows[b], acc_sh.at[dst_all.at[g0 + b]],
                                 ssem[b], add=True)
            for b in range(NBUF):
                pltpu.make_async_copy(rows[b], acc_sh.at[dst_all.at[g0 + b]],
                                      ssem[b]).wait()

                @pl.when(g0 + NBUF + b < G)
                def _():
                    pltpu.async_copy(p_hbm.at[src_all.at[g0 + NBUF + b]],
                                     rows[b], gsem[b])

            return carry

        lax.fori_loop(0, G // NBUF, rnd, 0)
        for b in range(G % NBUF):
            g = (G // NBUF) * NBUF + b
            pltpu.make_async_copy(p_hbm.at[src_all.at[g]], rows[b],
                                  gsem[b]).wait()
            pltpu.sync_copy(rows[b], acc_sh.at[dst_all.at[g]], add=True)
        plsc.subcore_barrier()

        @pl.when(c == 0)
        def _():
            pltpu.sync_copy(acc_sh.at[myrows], out0.at[myrows])

        @pl.when(c == 1)
        def _():
            pltpu.sync_copy(acc_sh.at[myrows], out1.at[myrows])

    return body


@functools.lru_cache(maxsize=None)
def _make_deg_scatter():
    _mesh = plsc.VectorSubcoreMesh(core_axis_name="c", subcore_axis_name="s")

    @functools.partial(
        pl.kernel,
        out_type=(jax.ShapeDtypeStruct((NACC, DEGW), _F32),
                  jax.ShapeDtypeStruct((NACC, DEGW), _F32)),
        mesh=_mesh,
        scratch_types=[
            pltpu.VMEM((G, K), jnp.int32),
            pltpu.VMEM((K, DEGW), _F32),
            pltpu.VMEM_SHARED((NACC, DEGW), _F32),
            pltpu.SemaphoreType.DMA,
        ],
        compiler_params=pltpu.CompilerParams(use_tc_tiling_on_sc=False),
    )
    def body(dst_hbm, ones_hbm, zer_hbm, out0, out1, dst_all, ones_v, acc_sh,
             sem):
        c = lax.axis_index("c")
        s = lax.axis_index("s")
        rows = pl.ds(s * ZR, ZR)
        pltpu.sync_copy(zer_hbm, acc_sh.at[rows])
        pltpu.sync_copy(ones_hbm, ones_v)
        wid = c * 16 + s
        pltpu.sync_copy(dst_hbm.at[pl.ds(wid * G, G)], dst_all)
        plsc.subcore_barrier()

        W = 8

        def rnd(r, carry):
            for b in range(W):
                pltpu.async_copy(ones_v, acc_sh.at[dst_all.at[r * W + b]],
                                 sem, add=True)
            for b in range(W):
                pltpu.make_async_copy(ones_v, acc_sh.at[dst_all.at[r * W + b]],
                                      sem).wait()
            return carry

        lax.fori_loop(0, G // W, rnd, 0)
        for g in range((G // W) * W, G):
            pltpu.sync_copy(ones_v, acc_sh.at[dst_all.at[g]], add=True)
        plsc.subcore_barrier()

        @pl.when(c == 0)
        def _():
            pltpu.sync_copy(acc_sh.at[rows], out0.at[rows])

        @pl.when(c == 1)
        def _():
            pltpu.sync_copy(acc_sh.at[rows], out1.at[rows])

    return body


def _dinv_of(d0, d1):
    return 1.0 / jnp.sqrt(d0[:, :1] + d1[:, :1] + 1.0)


def _tca_body(x_ref, w1_ref, d0_ref, d1_ref, p1_ref):
    di = _dinv_of(d0_ref[...], d1_ref[...])
    p1_ref[...] = di * _dot(x_ref[...], w1_ref[...])


def _tcb_body(a0_ref, a1_ref, p1_ref, b1_ref, d0_ref, d1_ref, p2_ref):
    di = _dinv_of(d0_ref[...], d1_ref[...])
    conv = di * (a0_ref[...] + a1_ref[...] + p1_ref[...]) + b1_ref[...]
    p2_ref[...] = di * jnp.maximum(conv, 0.0)


def _tcc_body(a0_ref, a1_ref, p2_ref, w2_ref, b2_ref, d0_ref, d1_ref, p3_ref):
    di = _dinv_of(d0_ref[...], d1_ref[...])
    m = di * (a0_ref[...] + a1_ref[...] + p2_ref[...])
    conv = _dot(m, w2_ref[...]) + b2_ref[...]
    p3_ref[...] = di * jnp.maximum(conv, 0.0)


def _tcd_body(a0_ref, a1_ref, p3_ref, w3_ref, b3_ref, d0_ref, d1_ref,
              batch_ref, cf_ref, wd_ref, bd_ref, wc1_ref, bc1_ref,
              wc2_ref, bc2_ref, wm1_ref, bm1_ref, wm2_ref, bm2_ref,
              wo_ref, bo_ref, out_ref, psum_s, cnt_s):
    i = pl.program_id(0)

    @pl.when(i == 0)
    def _():
        psum_s[...] = jnp.zeros_like(psum_s)
        cnt_s[...] = jnp.zeros_like(cnt_s)

    di = _dinv_of(d0_ref[...], d1_ref[...])
    m = di * (a0_ref[...] + a1_ref[...] + p3_ref[...])
    h3 = jnp.maximum(_dot(m, w3_ref[...]) + b3_ref[...], 0.0)
    oh = (batch_ref[...] == jax.lax.broadcasted_iota(jnp.int32, (1, B), 1))
    oh = oh.astype(_F32)
    tdot = lambda a, b: jax.lax.dot_general(
        a, b, (((0,), (0,)), ((), ())), precision=_HI,
        preferred_element_type=_F32)
    psum_s[...] += tdot(oh, h3)
    cnt_s[...] += tdot(oh, jnp.ones((RB, 1), _F32))

    @pl.when(i == _GRID - 1)
    def _():
        mean = psum_s[...] / jnp.maximum(cnt_s[...], 1.0)
        drug = _dot(mean, wd_ref[...]) + bd_ref[...]
        cellh = jnp.maximum(_dot(cf_ref[...], wc1_ref[...]) + bc1_ref[...],
                            0.0)
        cell = _dot(cellh, wc2_ref[...]) + bc2_ref[...]
        wm1 = wm1_ref[...]
        z = jnp.maximum(_dot(drug, wm1[:64]) + _dot(cell, wm1[64:])
                        + bm1_ref[...], 0.0)
        z = jnp.maximum(_dot(z, wm2_ref[...]) + bm2_ref[...], 0.0)
        out_ref[...] = _dot(z, wo_ref[...]) + bo_ref[...]


def _rows(shape):
    return pl.BlockSpec(shape, lambda i: (i, 0))


def _full(shape):
    return pl.BlockSpec(shape, lambda i: (0, 0))


_GRID = N // RB


def _tc_call(body, n_out_cols, in_specs):
    return pl.pallas_call(
        body,
        grid=(_GRID,),
        in_specs=in_specs,
        out_specs=_rows((RB, n_out_cols)),
        out_shape=jax.ShapeDtypeStruct((N, n_out_cols), _F32),
    )


def kernel(x, edge_index, batch, cell_features, W1, b1, W2, b2, W3, b3,
           Wd, bd, Wc1, bc1, Wc2, bc2, Wm1, bm1, Wm2, bm2, Wo, bo):
    src = edge_index[0].astype(jnp.int32)
    dst = edge_index[1].astype(jnp.int32)
    batch = batch.astype(jnp.int32)

    srcp = src.reshape(NTILES * G, K)
    dstp = dst.reshape(NTILES * G, K)

    z16 = jnp.zeros((ZR, DEGW), _F32)
    z64 = jnp.zeros((K, 64), _F32)
    z128 = jnp.zeros((K, 128), _F32)
    ones16 = jnp.ones((K, DEGW), _F32)

    d0, d1 = _make_deg_scatter()(dstp, ones16, z16)

    degspec = [_rows((RB, DEGW)), _rows((RB, DEGW))]
    p1 = _tc_call(_tca_body, 64,
                  [_rows((RB, 128)), _full((128, 64))] + degspec)(
                      x, W1, d0, d1)

    a0, a1 = _make_edge_scatter(64)(p1, srcp, dstp, z64)
    p2 = _tc_call(_tcb_body, 64,
                  [_rows((RB, 64)), _rows((RB, 64)), _rows((RB, 64)),
                   _full((1, 64))] + degspec)(
                      a0, a1, p1, b1.reshape(1, 64), d0, d1)

    a0, a1 = _make_edge_scatter(64)(p2, srcp, dstp, z64)
    p3 = _tc_call(_tcc_body, 128,
                  [_rows((RB, 64)), _rows((RB, 64)), _rows((RB, 64)),
                   _full((64, 128)), _full((1, 128))] + degspec)(
                      a0, a1, p2, W2, b2.reshape(1, 128), d0, d1)

    a0, a1 = _make_edge_scatter(128)(p3, srcp, dstp, z128)
    out = pl.pallas_call(
        _tcd_body,
        grid=(_GRID,),
        in_specs=[_rows((RB, 128)), _rows((RB, 128)), _rows((RB, 128)),
                  _full((128, 256)), _full((1, 256))] + degspec +
                 [_rows((RB, 1)), _full((B, 512)), _full((256, 64)),
                  _full((1, 64)), _full((512, 128)), _full((1, 128)),
                  _full((128, 64)), _full((1, 64)), _full((128, 64)),
                  _full((1, 64)), _full((64, 32)), _full((1, 32)),
                  _full((32, 1)), _full((1, 1))],
        out_specs=_full((B, 1)),
        out_shape=jax.ShapeDtypeStruct((B, 1), _F32),
        scratch_shapes=[pltpu.VMEM((B, 256), _F32), pltpu.VMEM((B, 1), _F32)],
    )(a0, a1, p3, W3, b3.reshape(1, 256), d0, d1, batch.reshape(N, 1),
      cell_features, Wd, bd.reshape(1, 64), Wc1, bc1.reshape(1, 128),
      Wc2, bc2.reshape(1, 64), Wm1, bm1.reshape(1, 64), Wm2,
      bm2.reshape(1, 32), Wo, bo.reshape(1, 1))

    return out.reshape(-1)


# speedup vs baseline: 35.9498x; 1.0321x over previous
import functools

import jax
import jax.numpy as jnp
from jax import lax
from jax.experimental import pallas as pl
import jax.experimental.pallas.tpu as pltpu
from jax.experimental.pallas import tpu_sc as plsc

N = 10000
E = 320000
B = 256
NTILES = 32
EP = E // NTILES
K = 80
G = EP // K
NACC = 10000
ZR = NACC // 16
DEGW = 8
RB = 2000
_F32 = jnp.float32
_HI = jax.lax.Precision.DEFAULT

def _dot(a, b):
    return jax.lax.dot_general(a, b, (((1,), (0,)), ((), ())),
                               precision=_HI, preferred_element_type=_F32)


def _es_dims(F):
    k = K if F <= 64 else 40
    return k, EP // k


@functools.lru_cache(maxsize=None)
def _make_edge_scatter(F):
    NBUF = 4 if F <= 64 else 6
    K_, G_ = _es_dims(F)
    _mesh = plsc.VectorSubcoreMesh(core_axis_name="c", subcore_axis_name="s")

    @functools.partial(
        pl.kernel,
        out_type=(jax.ShapeDtypeStruct((NACC, F), _F32),
                  jax.ShapeDtypeStruct((NACC, F), _F32)),
        mesh=_mesh,
        scratch_types=(
            [pltpu.VMEM((G_, K_), jnp.int32), pltpu.VMEM((G_, K_), jnp.int32)]
            + [pltpu.VMEM((K_, F), _F32) for _ in range(NBUF)]
            + [pltpu.VMEM_SHARED((NACC, F), _F32)]
            + [pltpu.SemaphoreType.DMA for _ in range(2 * NBUF)]
        ),
        compiler_params=pltpu.CompilerParams(use_tc_tiling_on_sc=False),
    )
    def body(p_hbm, src_hbm, dst_hbm, zer_hbm, out0, out1,
             src_all, dst_all, *bufs_acc_sems):
        rows = bufs_acc_sems[:NBUF]
        acc_sh = bufs_acc_sems[NBUF]
        gsem = bufs_acc_sems[NBUF + 1:2 * NBUF + 1]
        ssem = bufs_acc_sems[2 * NBUF + 1:]
        c = lax.axis_index("c")
        s = lax.axis_index("s")
        myrows = pl.ds(s * ZR, ZR)
        pltpu.sync_copy(zer_hbm, rows[0])
        for z in range(ZR // K_):
            pltpu.sync_copy(rows[0], acc_sh.at[pl.ds(s * ZR + z * K_, K_)])
        if ZR % K_:
            pltpu.sync_copy(rows[0].at[pl.ds(0, ZR % K_)],
                            acc_sh.at[pl.ds(s * ZR + (ZR // K_) * K_, ZR % K_)])
        wid = c * 16 + s
        pltpu.sync_copy(src_hbm.at[pl.ds(wid * G_, G_)], src_all)
        pltpu.sync_copy(dst_hbm.at[pl.ds(wid * G_, G_)], dst_all)
        plsc.subcore_barrier()

        for b in range(NBUF):
            pltpu.async_copy(p_hbm.at[src_all.at[b]], rows[b], gsem[b])

        def rnd(r, carry):
            g0 = r * NBUF
            for b in range(NBUF):
                pltpu.make_async_copy(p_hbm.at[src_all.at[g0 + b]], rows[b],
                                      gsem[b]).wait()
                pltpu.async_copy(rows[b], acc_sh.at[dst_all.at[g0 + b]],
                                 ssem[b], add=True)
            for b in range(NBUF):
                pltpu.make_async_copy(rows[b], acc_sh.at[dst_all.at[g0 + b]],
                                      ssem[b]).wait()

                @pl.when(g0 + NBUF + b < G_)
                def _():
                    pltpu.async_copy(p_hbm.at[src_all.at[g0 + NBUF + b]],
                                     rows[b], gsem[b])

            return carry

        lax.fori_loop(0, G_ // NBUF, rnd, 0)
        for b in range(G_ % NBUF):
            g = (G_ // NBUF) * NBUF + b
            pltpu.make_async_copy(p_hbm.at[src_all.at[g]], rows[b],
                                  gsem[b]).wait()
            pltpu.sync_copy(rows[b], acc_sh.at[dst_all.at[g]], add=True)
        plsc.subcore_barrier()

        @pl.when(c == 0)
        def _():
            pltpu.sync_copy(acc_sh.at[myrows], out0.at[myrows])

        @pl.when(c == 1)
        def _():
            pltpu.sync_copy(acc_sh.at[myrows], out1.at[myrows])

    return body


@functools.lru_cache(maxsize=None)
def _make_deg_scatter():
    _mesh = plsc.VectorSubcoreMesh(core_axis_name="c", subcore_axis_name="s")

    @functools.partial(
        pl.kernel,
        out_type=(jax.ShapeDtypeStruct((NACC, DEGW), _F32),
                  jax.ShapeDtypeStruct((NACC, DEGW), _F32)),
        mesh=_mesh,
        scratch_types=[
            pltpu.VMEM((G, K), jnp.int32),
            pltpu.VMEM((K, DEGW), _F32),
            pltpu.VMEM_SHARED((NACC, DEGW), _F32),
            pltpu.SemaphoreType.DMA,
        ],
        compiler_params=pltpu.CompilerParams(use_tc_tiling_on_sc=False),
    )
    def body(dst_hbm, ones_hbm, zer_hbm, out0, out1, dst_all, ones_v, acc_sh,
             sem):
        c = lax.axis_index("c")
        s = lax.axis_index("s")
        rows = pl.ds(s * ZR, ZR)
        pltpu.sync_copy(zer_hbm, acc_sh.at[rows])
        pltpu.sync_copy(ones_hbm, ones_v)
        wid = c * 16 + s
        pltpu.sync_copy(dst_hbm.at[pl.ds(wid * G, G)], dst_all)
        plsc.subcore_barrier()

        W = 8

        def rnd(r, carry):
            for b in range(W):
                pltpu.async_copy(ones_v, acc_sh.at[dst_all.at[r * W + b]],
                                 sem, add=True)
            for b in range(W):
                pltpu.make_async_copy(ones_v, acc_sh.at[dst_all.at[r * W + b]],
                                      sem).wait()
            return carry

        lax.fori_loop(0, G // W, rnd, 0)
        for g in range((G // W) * W, G):
            pltpu.sync_copy(ones_v, acc_sh.at[dst_all.at[g]], add=True)
        plsc.subcore_barrier()

        @pl.when(c == 0)
        def _():
            pltpu.sync_copy(acc_sh.at[rows], out0.at[rows])

        @pl.when(c == 1)
        def _():
            pltpu.sync_copy(acc_sh.at[rows], out1.at[rows])

    return body


def _dinv_of(d0, d1):
    return 1.0 / jnp.sqrt(d0[:, :1] + d1[:, :1] + 1.0)


def _tca_body(x_ref, w1_ref, d0_ref, d1_ref, p1_ref):
    di = _dinv_of(d0_ref[...], d1_ref[...])
    p1_ref[...] = di * _dot(x_ref[...], w1_ref[...])


def _tcb_body(a0_ref, a1_ref, p1_ref, b1_ref, d0_ref, d1_ref, p2_ref):
    di = _dinv_of(d0_ref[...], d1_ref[...])
    conv = di * (a0_ref[...] + a1_ref[...] + p1_ref[...]) + b1_ref[...]
    p2_ref[...] = di * jnp.maximum(conv, 0.0)


def _tcc_body(a0_ref, a1_ref, p2_ref, w2_ref, b2_ref, d0_ref, d1_ref, p3_ref):
    di = _dinv_of(d0_ref[...], d1_ref[...])
    m = di * (a0_ref[...] + a1_ref[...] + p2_ref[...])
    conv = _dot(m, w2_ref[...]) + b2_ref[...]
    p3_ref[...] = di * jnp.maximum(conv, 0.0)


def _tcd_body(a0_ref, a1_ref, p3_ref, w3_ref, b3_ref, d0_ref, d1_ref,
              batch_ref, cf_ref, wd_ref, bd_ref, wc1_ref, bc1_ref,
              wc2_ref, bc2_ref, wm1_ref, bm1_ref, wm2_ref, bm2_ref,
              wo_ref, bo_ref, out_ref, psum_s, cnt_s):
    i = pl.program_id(0)

    @pl.when(i == 0)
    def _():
        psum_s[...] = jnp.zeros_like(psum_s)
        cnt_s[...] = jnp.zeros_like(cnt_s)

    di = _dinv_of(d0_ref[...], d1_ref[...])
    m = di * (a0_ref[...] + a1_ref[...] + p3_ref[...])
    h3 = jnp.maximum(_dot(m, w3_ref[...]) + b3_ref[...], 0.0)
    oh = (batch_ref[...] == jax.lax.broadcasted_iota(jnp.int32, (1, B), 1))
    oh = oh.astype(_F32)
    tdot = lambda a, b: jax.lax.dot_general(
        a, b, (((0,), (0,)), ((), ())), precision=_HI,
        preferred_element_type=_F32)
    psum_s[...] += tdot(oh, h3)
    cnt_s[...] += tdot(oh, jnp.ones((RB, 1), _F32))

    @pl.when(i == _GRID - 1)
    def _():
        mean = psum_s[...] / jnp.maximum(cnt_s[...], 1.0)
        drug = _dot(mean, wd_ref[...]) + bd_ref[...]
        cellh = jnp.maximum(_dot(cf_ref[...], wc1_ref[...]) + bc1_ref[...],
                            0.0)
        cell = _dot(cellh, wc2_ref[...]) + bc2_ref[...]
        wm1 = wm1_ref[...]
        z = jnp.maximum(_dot(drug, wm1[:64]) + _dot(cell, wm1[64:])
                        + bm1_ref[...], 0.0)
        z = jnp.maximum(_dot(z, wm2_ref[...]) + bm2_ref[...], 0.0)
        out_ref[...] = _dot(z, wo_ref[...]) + bo_ref[...]


def _rows(shape):
    return pl.BlockSpec(shape, lambda i: (i, 0))


def _full(shape):
    return pl.BlockSpec(shape, lambda i: (0, 0))


_GRID = N // RB


def _tc_call(body, n_out_cols, in_specs):
    return pl.pallas_call(
        body,
        grid=(_GRID,),
        in_specs=in_specs,
        out_specs=_rows((RB, n_out_cols)),
        out_shape=jax.ShapeDtypeStruct((N, n_out_cols), _F32),
    )


def kernel(x, edge_index, batch, cell_features, W1, b1, W2, b2, W3, b3,
           Wd, bd, Wc1, bc1, Wc2, bc2, Wm1, bm1, Wm2, bm2, Wo, bo):
    src = edge_index[0].astype(jnp.int32)
    dst = edge_index[1].astype(jnp.int32)
    batch = batch.astype(jnp.int32)

    k64, g64 = _es_dims(64)
    k128, g128 = _es_dims(128)
    srcp = src.reshape(NTILES * g64, k64)
    dstp = dst.reshape(NTILES * g64, k64)
    srcp128 = src.reshape(NTILES * g128, k128)
    dstp128 = dst.reshape(NTILES * g128, k128)

    z16 = jnp.zeros((ZR, DEGW), _F32)
    z64 = jnp.zeros((k64, 64), _F32)
    z128 = jnp.zeros((k128, 128), _F32)
    ones16 = jnp.ones((K, DEGW), _F32)

    d0, d1 = _make_deg_scatter()(dstp, ones16, z16)

    degspec = [_rows((RB, DEGW)), _rows((RB, DEGW))]
    p1 = _tc_call(_tca_body, 64,
                  [_rows((RB, 128)), _full((128, 64))] + degspec)(
                      x, W1, d0, d1)

    a0, a1 = _make_edge_scatter(64)(p1, srcp, dstp, z64)
    p2 = _tc_call(_tcb_body, 64,
                  [_rows((RB, 64)), _rows((RB, 64)), _rows((RB, 64)),
                   _full((1, 64))] + degspec)(
                      a0, a1, p1, b1.reshape(1, 64), d0, d1)

    a0, a1 = _make_edge_scatter(64)(p2, srcp, dstp, z64)
    p3 = _tc_call(_tcc_body, 128,
                  [_rows((RB, 64)), _rows((RB, 64)), _rows((RB, 64)),
                   _full((64, 128)), _full((1, 128))] + degspec)(
                      a0, a1, p2, W2, b2.reshape(1, 128), d0, d1)

    a0, a1 = _make_edge_scatter(128)(p3, srcp128, dstp128, z128)
    out = pl.pallas_call(
        _tcd_body,
        grid=(_GRID,),
        in_specs=[_rows((RB, 128)), _rows((RB, 128)), _rows((RB, 128)),
                  _full((128, 256)), _full((1, 256))] + degspec +
                 [_rows((RB, 1)), _full((B, 512)), _full((256, 64)),
                  _full((1, 64)), _full((512, 128)), _full((1, 128)),
                  _full((128, 64)), _full((1, 64)), _full((128, 64)),
                  _full((1, 64)), _full((64, 32)), _full((1, 32)),
                  _full((32, 1)), _full((1, 1))],
        out_specs=_full((B, 1)),
        out_shape=jax.ShapeDtypeStruct((B, 1), _F32),
        scratch_shapes=[pltpu.VMEM((B, 256), _F32), pltpu.VMEM((B, 1), _F32)],
    )(a0, a1, p3, W3, b3.reshape(1, 256), d0, d1, batch.reshape(N, 1),
      cell_features, Wd, bd.reshape(1, 64), Wc1, bc1.reshape(1, 128),
      Wc2, bc2.reshape(1, 64), Wm1, bm1.reshape(1, 64), Wm2,
      bm2.reshape(1, 32), Wo, bo.reshape(1, 1))

    return out.reshape(-1)
